# baseline (speedup 1.0000x reference)
import jax
import jax.numpy as jnp
import numpy as np
from jax.experimental import pallas as pl

D = 64
NI = 1000001
LSEQ = 200
NNODES = 65536
NE = 131072
NB = 1024
STEP = 2


def _seg_mean(vals, ids, num):
    s = jax.ops.segment_sum(vals, ids, num)
    c = jax.ops.segment_sum(jnp.ones((vals.shape[0],), vals.dtype), ids, num)
    return s / jnp.maximum(c, 1.0)[:, None]


def _layer_norm_sum(x):
    y = x - jnp.mean(x, -1, keepdims=True)
    return y / jnp.sqrt(jnp.sum(y ** 2, -1, keepdims=True))


def kernel(params, x, edge_index, batch, alias_inputs, item_seq_len):
    p = params
    hidden = p['item_emb'][x]
    star = _seg_mean(hidden, batch, NB)
    src, dst = edge_index[0], edge_index[1]
    for _ in range(STEP):
        xin = hidden @ p['W_in'] + p['b_in']
        input_in = _seg_mean(xin[src], dst, NNODES)
        xout = hidden @ p['W_out'] + p['b_out']
        input_out = _seg_mean(xout[dst], src, NNODES)
        inputs = jnp.concatenate([input_in, input_out], -1)
        gi = inputs @ p['W_ih'] + p['b_ih']
        gh = hidden @ p['W_hh'] + p['b_hh']
        i_r, i_i, i_n = jnp.split(gi, 3, -1)
        h_r, h_i, h_n = jnp.split(gh, 3, -1)
        reset_gate = jax.nn.sigmoid(i_r + h_r)
        input_gate = jax.nn.sigmoid(i_i + h_i)
        new_gate = jnp.tanh(i_n + reset_gate * h_n)
        hidden = (1 - input_gate) * hidden + input_gate * new_gate
        star_rep = star[batch]
        sim = jnp.sum(hidden * star_rep, -1, keepdims=True) / np.sqrt(D)
        alpha = jax.nn.sigmoid(sim)
        hidden = (1 - alpha) * hidden + alpha * star_rep
        s = jnp.sum(hidden * star_rep, -1)
        m = jax.ops.segment_max(s, batch, NB)
        e = jnp.exp(s - m[batch])
        den = jax.ops.segment_sum(e, batch, NB)
        w = e / den[batch]
        star = jax.ops.segment_sum(w[:, None] * hidden, batch, NB)
    mask = alias_inputs > 0
    seq_hidden = hidden[alias_inputs]
    pos = jnp.broadcast_to(p['pos_emb'][None, :, :], (NB, LSEQ, D))
    lenmask = jnp.arange(LSEQ)[None, :] < item_seq_len[:, None]
    pos = pos * lenmask[..., None]
    seq_hidden = seq_hidden + pos
    isl = jnp.maximum(item_seq_len, 1)
    ht = seq_hidden[jnp.arange(NB), isl - 1]
    q1 = (ht @ p['W1'] + p['b1'])[:, None, :]
    q2 = seq_hidden @ p['W2'] + p['b2']
    q3 = (star @ p['W3'] + p['b3'])[:, None, :]
    alpha = jax.nn.sigmoid(q1 + q2 + q3) @ p['W4']
    a = jnp.sum(alpha * seq_hidden * mask[..., None].astype(jnp.float32), 1)
    seq_output = jnp.concatenate([a, ht], 1) @ p['Wt'] + p['bt']
    return _layer_norm_sum(seq_output)


# speedup vs baseline: 1.6793x; 1.6793x over previous
import functools

import jax
import jax.numpy as jnp
import numpy as np
from jax import lax
from jax.experimental import pallas as pl
from jax.experimental.pallas import tpu as pltpu
from jax.experimental.pallas import tpu_sc as plsc

D = 64
NI = 1000001
LSEQ = 200
NNODES = 65536
NE = 131072
NB = 1024
STEP = 2

NC = 2
NS = 16
NW = NC * NS

_MESH = plsc.VectorSubcoreMesh(core_axis_name="c", subcore_axis_name="s")

QROWS = NNODES // 4
ACC_ROWS = QROWS + 128
ROWS_PER_TILE_Z = ACC_ROWS // NS


def _zero_vmem_rows(buf, nrows):
    zv = jnp.zeros((16,), jnp.float32)

    def body2(i, _):
        r = i // (D // 16)
        k = i % (D // 16)
        buf[r, pl.ds(k * 16, 16)] = zv
        return None

    lax.fori_loop(0, nrows * D // 16, body2, None)


def _emb_gather_body(item_emb, x, batch3, hidden, star_part, idx_v, rows_v,
                     bidx_v, zbuf, star_acc, sem):
    c = lax.axis_index("c")
    s = lax.axis_index("s")
    wid = s * NC + c

    _zero_vmem_rows(zbuf, 128)
    pltpu.sync_copy(zbuf.at[pl.ds(0, 68)], star_acc.at[pl.ds(s * 68, 68)])
    plsc.subcore_barrier()

    rows_per_w = NNODES // NW
    def chunk(k, _):
        base = wid * rows_per_w + k * 1024
        pltpu.sync_copy(x.at[pl.ds(base, 1024)], idx_v)
        pltpu.async_copy(item_emb.at[idx_v], rows_v, sem).wait()
        pltpu.sync_copy(rows_v, hidden.at[pl.ds(base, 1024)])
        pltpu.sync_copy(batch3.at[pl.ds(base // 128, 8)], bidx_v)

        def seg(j, _):
            pltpu.sync_copy(rows_v.at[pl.ds(j * 128, 128)],
                            star_acc.at[bidx_v.at[j, 0]], add=True)
            return None
        lax.fori_loop(0, 8, seg, None)
        return None
    lax.fori_loop(0, 2, chunk, None)

    plsc.subcore_barrier()
    pltpu.sync_copy(star_acc.at[pl.ds(s * 64, 64)],
                    star_part.at[c, pl.ds(s * 64, 64)])


def _emb_gather(item_emb, x, batch3):
    f = pl.kernel(
        _emb_gather_body,
        out_type=[
            jax.ShapeDtypeStruct((NNODES, D), jnp.float32),
            jax.ShapeDtypeStruct((NC, NB, D), jnp.float32),
        ],
        mesh=_MESH,
        compiler_params=pltpu.CompilerParams(use_tc_tiling_on_sc=False),
        scratch_types=[
            pltpu.VMEM((1024,), jnp.int32),
            pltpu.VMEM((1024, D), jnp.float32),
            pltpu.VMEM((8, 1, 128), jnp.int32),
            pltpu.VMEM((128, D), jnp.float32),
            pltpu.VMEM_SHARED((NS * 68, D), jnp.float32),
            pltpu.SemaphoreType.DMA,
        ],
    )
    return f(item_emb, x, batch3)


def _hist_body(src3, dst3, batch3, cnt_out, cnt_in, cnt_b,
               idx_v, ones_v, zflat, hsrc, hdst, hb):
    c = lax.axis_index("c")
    s = lax.axis_index("s")

    def fill16(i, _):
        ones_v[pl.ds(i * 16, 16)] = jnp.ones((16,), jnp.float32)
        return None
    lax.fori_loop(0, 8, fill16, None)
    zv = jnp.zeros((16,), jnp.float32)
    def z16(i, _):
        zflat[pl.ds(i * 16, 16)] = zv
        return None
    lax.fori_loop(0, 4096 // 16, z16, None)
    pltpu.sync_copy(zflat.at[pl.ds(0, 4096)], hsrc.at[pl.ds(s * 4096, 4096)])
    pltpu.sync_copy(zflat.at[pl.ds(0, 4096)], hdst.at[pl.ds(s * 4096, 4096)])
    pltpu.sync_copy(zflat.at[pl.ds(0, 64)], hb.at[pl.ds(s * 64, 64)])
    plsc.subcore_barrier()

    e_per_t = NE // NC // NS
    n_per_t = NNODES // NC // NS

    def do_hist(arr3, acc, per_t, base_rows):
        def chunk(k, _):
            row = base_rows + k
            pltpu.sync_copy(arr3.at[pl.ds(row, 1)], idx_v)
            pltpu.sync_copy(ones_v, acc.at[idx_v.at[0, 0]], add=True)
            return None
        lax.fori_loop(0, per_t // 128, chunk, None)

    base_e = (c * NS + s) * (e_per_t // 128)
    base_n = (c * NS + s) * (n_per_t // 128)
    do_hist(src3, hsrc, e_per_t, base_e)
    do_hist(dst3, hdst, e_per_t, base_e)
    do_hist(batch3, hb, n_per_t, base_n)
    plsc.subcore_barrier()

    per = NNODES // NS
    pltpu.sync_copy(hsrc.at[pl.ds(s * per, per)], cnt_out.at[c, pl.ds(s * per, per)])
    pltpu.sync_copy(hdst.at[pl.ds(s * per, per)], cnt_in.at[c, pl.ds(s * per, per)])
    pltpu.sync_copy(hb.at[pl.ds(s * 64, 64)], cnt_b.at[c, pl.ds(s * 64, 64)])


def _hist(src3, dst3, batch3):
    f = pl.kernel(
        _hist_body,
        out_type=[
            jax.ShapeDtypeStruct((NC, NNODES), jnp.float32),
            jax.ShapeDtypeStruct((NC, NNODES), jnp.float32),
            jax.ShapeDtypeStruct((NC, NB), jnp.float32),
        ],
        mesh=_MESH,
        compiler_params=pltpu.CompilerParams(use_tc_tiling_on_sc=False),
        scratch_types=[
            pltpu.VMEM((1, 1, 128), jnp.int32),
            pltpu.VMEM((128,), jnp.float32),
            pltpu.VMEM((4096,), jnp.float32),
            pltpu.VMEM_SHARED((NNODES,), jnp.float32),
            pltpu.VMEM_SHARED((NNODES,), jnp.float32),
            pltpu.VMEM_SHARED((NS * 64,), jnp.float32),
        ],
    )
    return f(src3, dst3, batch3)


def _edge_agg_body(hidden, gidx, sidx3, out, idx_v, sidx_v, rows_v, zbuf,
                   acc, sem):
    c = lax.axis_index("c")
    s = lax.axis_index("s")

    _zero_vmem_rows(zbuf, 128)

    def quarter(qi, _):
        q = c * 2 + qi
        qbase = q * QROWS

        def zr(i, _):
            pltpu.sync_copy(
                zbuf.at[pl.ds(0, 128)],
                acc.at[pl.ds(s * ROWS_PER_TILE_Z + i * 128, 128)])
            return None
        lax.fori_loop(0, ROWS_PER_TILE_Z // 128, zr, None)
        pltpu.sync_copy(
            zbuf.at[pl.ds(0, ROWS_PER_TILE_Z % 128)],
            acc.at[pl.ds(s * ROWS_PER_TILE_Z
                         + (ROWS_PER_TILE_Z // 128) * 128,
                         ROWS_PER_TILE_Z % 128)])
        plsc.subcore_barrier()

        e_per_t = NE // NS
        def chunk(k, _):
            base = s * e_per_t + k * 512
            pltpu.sync_copy(gidx.at[pl.ds(base, 512)], idx_v)
            pltpu.sync_copy(sidx3.at[pl.ds(base // 128, 4)], sidx_v)
            pltpu.async_copy(hidden.at[idx_v], rows_v, sem).wait()

            def remap(t, _):
                jj = t // 8
                kk = t % 8
                v = sidx_v[jj, 0, pl.ds(kk * 16, 16)]
                rel = v - qbase
                ok = (rel >= 0) & (rel < QROWS)
                sidx_v[jj, 0, pl.ds(kk * 16, 16)] = jnp.where(
                    ok, rel, QROWS)
                return None
            lax.fori_loop(0, 32, remap, None)

            def seg(j, _):
                pltpu.sync_copy(rows_v.at[pl.ds(j * 128, 128)],
                                acc.at[sidx_v.at[j, 0]], add=True)
                return None
            lax.fori_loop(0, 4, seg, None)
            return None
        lax.fori_loop(0, e_per_t // 512, chunk, None)

        plsc.subcore_barrier()
        rpt = QROWS // NS
        def wout(i, _):
            pltpu.sync_copy(acc.at[pl.ds(s * rpt + i * 128, 128)],
                            out.at[pl.ds(qbase + s * rpt + i * 128, 128)])
            return None
        lax.fori_loop(0, rpt // 128, wout, None)
        plsc.subcore_barrier()
        return None

    lax.fori_loop(0, 2, quarter, None)


def _edge_agg(hidden, gidx, sidx3):
    f = pl.kernel(
        _edge_agg_body,
        out_type=jax.ShapeDtypeStruct((NNODES, D), jnp.float32),
        mesh=_MESH,
        compiler_params=pltpu.CompilerParams(use_tc_tiling_on_sc=False),
        scratch_types=[
            pltpu.VMEM((512,), jnp.int32),
            pltpu.VMEM((4, 1, 128), jnp.int32),
            pltpu.VMEM((512, D), jnp.float32),
            pltpu.VMEM((128, D), jnp.float32),
            pltpu.VMEM_SHARED((ACC_ROWS, D), jnp.float32),
            pltpu.SemaphoreType.DMA,
        ],
    )
    return f(hidden, gidx, sidx3)


def _seq_gather_body(hidden, alias_flat, seq, idx_v, rows_v, sem):
    c = lax.axis_index("c")
    s = lax.axis_index("s")
    wid = s * NC + c
    per_w = (NB * LSEQ) // NW

    def chunk(k, _):
        base = wid * per_w + k * 800
        pltpu.sync_copy(alias_flat.at[pl.ds(base, 800)], idx_v)
        pltpu.async_copy(hidden.at[idx_v], rows_v, sem).wait()
        pltpu.sync_copy(rows_v, seq.at[pl.ds(base, 800)])
        return None
    lax.fori_loop(0, per_w // 800, chunk, None)


def _seq_gather(hidden, alias_flat):
    f = pl.kernel(
        _seq_gather_body,
        out_type=jax.ShapeDtypeStruct((NB * LSEQ, D), jnp.float32),
        mesh=_MESH,
        compiler_params=pltpu.CompilerParams(use_tc_tiling_on_sc=False),
        scratch_types=[
            pltpu.VMEM((800,), jnp.int32),
            pltpu.VMEM((800, D), jnp.float32),
            pltpu.SemaphoreType.DMA,
        ],
    )
    return f(hidden, alias_flat)


def _layer_norm_sum(x):
    y = x - jnp.mean(x, -1, keepdims=True)
    return y / jnp.sqrt(jnp.sum(y ** 2, -1, keepdims=True))


def kernel(params, x, edge_index, batch, alias_inputs, item_seq_len):
    p = params
    src, dst = edge_index[0], edge_index[1]
    src3 = src.reshape(NE // 128, 1, 128)
    dst3 = dst.reshape(NE // 128, 1, 128)
    batch3 = batch.reshape(NNODES // 128, 1, 128)

    hidden, star_part = _emb_gather(p['item_emb'], x, batch3)
    cnt_out2, cnt_in2, cnt_b2 = _hist(src3, dst3, batch3)
    cnt_in = (cnt_in2[0] + cnt_in2[1])
    cnt_out = (cnt_out2[0] + cnt_out2[1])
    cnt_b = (cnt_b2[0] + cnt_b2[1])

    star = (star_part[0] + star_part[1]) / jnp.maximum(cnt_b, 1.0)[:, None]

    rin = (1.0 / jnp.maximum(cnt_in, 1.0))[:, None]
    rout = (1.0 / jnp.maximum(cnt_out, 1.0))[:, None]
    min_ = (cnt_in > 0).astype(jnp.float32)[:, None]
    mout_ = (cnt_out > 0).astype(jnp.float32)[:, None]

    for _ in range(STEP):
        agg_in = _edge_agg(hidden, src, dst3)
        agg_out = _edge_agg(hidden, dst, src3)
        input_in = (agg_in * rin) @ p['W_in'] + p['b_in'] * min_
        input_out = (agg_out * rout) @ p['W_out'] + p['b_out'] * mout_
        inputs = jnp.concatenate([input_in, input_out], -1)
        gi = inputs @ p['W_ih'] + p['b_ih']
        gh = hidden @ p['W_hh'] + p['b_hh']
        i_r, i_i, i_n = jnp.split(gi, 3, -1)
        h_r, h_i, h_n = jnp.split(gh, 3, -1)
        reset_gate = jax.nn.sigmoid(i_r + h_r)
        input_gate = jax.nn.sigmoid(i_i + h_i)
        new_gate = jnp.tanh(i_n + reset_gate * h_n)
        hidden = (1 - input_gate) * hidden + input_gate * new_gate
        star_rep = star[batch]
        sim = jnp.sum(hidden * star_rep, -1, keepdims=True) / np.sqrt(D)
        alpha = jax.nn.sigmoid(sim)
        hidden = (1 - alpha) * hidden + alpha * star_rep
        s = jnp.sum(hidden * star_rep, -1)
        e = jnp.exp(s)
        den = jax.ops.segment_sum(e, batch, NB, indices_are_sorted=True)
        w = e / den[batch]
        star = jax.ops.segment_sum(w[:, None] * hidden, batch, NB,
                                   indices_are_sorted=True)

    mask = alias_inputs > 0
    seq_hidden = _seq_gather(hidden, alias_inputs.reshape(-1)).reshape(
        NB, LSEQ, D)
    pos = jnp.broadcast_to(p['pos_emb'][None, :, :], (NB, LSEQ, D))
    lenmask = jnp.arange(LSEQ)[None, :] < item_seq_len[:, None]
    pos = pos * lenmask[..., None]
    seq_hidden = seq_hidden + pos
    isl = jnp.maximum(item_seq_len, 1)
    ht = seq_hidden[jnp.arange(NB), isl - 1]
    q1 = (ht @ p['W1'] + p['b1'])[:, None, :]
    q2 = seq_hidden @ p['W2'] + p['b2']
    q3 = (star @ p['W3'] + p['b3'])[:, None, :]
    alpha = jax.nn.sigmoid(q1 + q2 + q3) @ p['W4']
    a = jnp.sum(alpha * seq_hidden * mask[..., None].astype(jnp.float32), 1)
    seq_output = jnp.concatenate([a, ht], 1) @ p['Wt'] + p['bt']
    return _layer_norm_sum(seq_output)


# speedup vs baseline: 1.9916x; 1.1860x over previous
import functools

import jax
import jax.numpy as jnp
import numpy as np
from jax import lax
from jax.experimental import pallas as pl
from jax.experimental.pallas import tpu as pltpu
from jax.experimental.pallas import tpu_sc as plsc

D = 64
NI = 1000001
LSEQ = 200
NNODES = 65536
NE = 131072
NB = 1024
STEP = 2

NC = 2
NS = 16
NW = NC * NS

_MESH = plsc.VectorSubcoreMesh(core_axis_name="c", subcore_axis_name="s")

QROWS = NNODES // 4
ACC_ROWS = QROWS + 128
ROWS_PER_TILE_Z = ACC_ROWS // NS


def _zero_vmem_rows(buf, nrows):
    zv = jnp.zeros((16,), jnp.float32)

    def body2(i, _):
        r = i // (D // 16)
        k = i % (D // 16)
        buf[r, pl.ds(k * 16, 16)] = zv
        return None

    lax.fori_loop(0, nrows * D // 16, body2, None)


def _emb_gather_body(item_emb, x, batch3, hidden, star_part, idx_v, rows_v,
                     bidx_v, zbuf, star_acc, sem):
    c = lax.axis_index("c")
    s = lax.axis_index("s")
    wid = s * NC + c

    _zero_vmem_rows(zbuf, 128)
    pltpu.sync_copy(zbuf.at[pl.ds(0, 68)], star_acc.at[pl.ds(s * 68, 68)])
    plsc.subcore_barrier()

    rows_per_w = NNODES // NW
    def chunk(k, _):
        base = wid * rows_per_w + k * 1024
        pltpu.sync_copy(x.at[pl.ds(base, 1024)], idx_v)
        pltpu.async_copy(item_emb.at[idx_v], rows_v, sem).wait()
        pltpu.sync_copy(rows_v, hidden.at[pl.ds(base, 1024)])
        pltpu.sync_copy(batch3.at[pl.ds(base // 128, 8)], bidx_v)

        def seg(j, _):
            pltpu.sync_copy(rows_v.at[pl.ds(j * 128, 128)],
                            star_acc.at[bidx_v.at[j, 0]], add=True)
            return None
        lax.fori_loop(0, 8, seg, None)
        return None
    lax.fori_loop(0, 2, chunk, None)

    plsc.subcore_barrier()
    pltpu.sync_copy(star_acc.at[pl.ds(s * 64, 64)],
                    star_part.at[c, pl.ds(s * 64, 64)])


def _emb_gather(item_emb, x, batch3):
    f = pl.kernel(
        _emb_gather_body,
        out_type=[
            jax.ShapeDtypeStruct((NNODES, D), jnp.float32),
            jax.ShapeDtypeStruct((NC, NB, D), jnp.float32),
        ],
        mesh=_MESH,
        compiler_params=pltpu.CompilerParams(use_tc_tiling_on_sc=False),
        scratch_types=[
            pltpu.VMEM((1024,), jnp.int32),
            pltpu.VMEM((1024, D), jnp.float32),
            pltpu.VMEM((8, 1, 128), jnp.int32),
            pltpu.VMEM((128, D), jnp.float32),
            pltpu.VMEM_SHARED((NS * 68, D), jnp.float32),
            pltpu.SemaphoreType.DMA,
        ],
    )
    return f(item_emb, x, batch3)


def _hist_body(src3, dst3, batch3, cnt_out, cnt_in, cnt_b,
               idx_v, ones_v, zflat, hsrc, hdst, hb):
    c = lax.axis_index("c")
    s = lax.axis_index("s")

    def fill16(i, _):
        ones_v[pl.ds(i * 16, 16)] = jnp.ones((16,), jnp.float32)
        return None
    lax.fori_loop(0, 8, fill16, None)
    zv = jnp.zeros((16,), jnp.float32)
    def z16(i, _):
        zflat[pl.ds(i * 16, 16)] = zv
        return None
    lax.fori_loop(0, 4096 // 16, z16, None)
    pltpu.sync_copy(zflat.at[pl.ds(0, 4096)], hsrc.at[pl.ds(s * 4096, 4096)])
    pltpu.sync_copy(zflat.at[pl.ds(0, 4096)], hdst.at[pl.ds(s * 4096, 4096)])
    pltpu.sync_copy(zflat.at[pl.ds(0, 64)], hb.at[pl.ds(s * 64, 64)])
    plsc.subcore_barrier()

    e_per_t = NE // NC // NS
    n_per_t = NNODES // NC // NS

    def do_hist(arr3, acc, per_t, base_rows):
        def chunk(k, _):
            row = base_rows + k
            pltpu.sync_copy(arr3.at[pl.ds(row, 1)], idx_v)
            pltpu.sync_copy(ones_v, acc.at[idx_v.at[0, 0]], add=True)
            return None
        lax.fori_loop(0, per_t // 128, chunk, None)

    base_e = (c * NS + s) * (e_per_t // 128)
    base_n = (c * NS + s) * (n_per_t // 128)
    do_hist(src3, hsrc, e_per_t, base_e)
    do_hist(dst3, hdst, e_per_t, base_e)
    do_hist(batch3, hb, n_per_t, base_n)
    plsc.subcore_barrier()

    per = NNODES // NS
    pltpu.sync_copy(hsrc.at[pl.ds(s * per, per)], cnt_out.at[c, pl.ds(s * per, per)])
    pltpu.sync_copy(hdst.at[pl.ds(s * per, per)], cnt_in.at[c, pl.ds(s * per, per)])
    pltpu.sync_copy(hb.at[pl.ds(s * 64, 64)], cnt_b.at[c, pl.ds(s * 64, 64)])


def _hist(src3, dst3, batch3):
    f = pl.kernel(
        _hist_body,
        out_type=[
            jax.ShapeDtypeStruct((NC, NNODES), jnp.float32),
            jax.ShapeDtypeStruct((NC, NNODES), jnp.float32),
            jax.ShapeDtypeStruct((NC, NB), jnp.float32),
        ],
        mesh=_MESH,
        compiler_params=pltpu.CompilerParams(use_tc_tiling_on_sc=False),
        scratch_types=[
            pltpu.VMEM((1, 1, 128), jnp.int32),
            pltpu.VMEM((128,), jnp.float32),
            pltpu.VMEM((4096,), jnp.float32),
            pltpu.VMEM_SHARED((NNODES,), jnp.float32),
            pltpu.VMEM_SHARED((NNODES,), jnp.float32),
            pltpu.VMEM_SHARED((NS * 64,), jnp.float32),
        ],
    )
    return f(src3, dst3, batch3)


def _edge_agg_body(hidden, gidx, sidx3, out, idx_v, sidx_v, rows_v, zbuf,
                   acc, sem):
    c = lax.axis_index("c")
    s = lax.axis_index("s")

    _zero_vmem_rows(zbuf, 128)

    def quarter(qi, _):
        q = c * 2 + qi
        qbase = q * QROWS

        def zr(i, _):
            pltpu.sync_copy(
                zbuf.at[pl.ds(0, 128)],
                acc.at[pl.ds(s * ROWS_PER_TILE_Z + i * 128, 128)])
            return None
        lax.fori_loop(0, ROWS_PER_TILE_Z // 128, zr, None)
        pltpu.sync_copy(
            zbuf.at[pl.ds(0, ROWS_PER_TILE_Z % 128)],
            acc.at[pl.ds(s * ROWS_PER_TILE_Z
                         + (ROWS_PER_TILE_Z // 128) * 128,
                         ROWS_PER_TILE_Z % 128)])
        plsc.subcore_barrier()

        e_per_t = NE // NS
        def chunk(k, _):
            base = s * e_per_t + k * 512
            pltpu.sync_copy(gidx.at[pl.ds(base, 512)], idx_v)
            pltpu.sync_copy(sidx3.at[pl.ds(base // 128, 4)], sidx_v)
            pltpu.async_copy(hidden.at[idx_v], rows_v, sem).wait()

            def remap(t, _):
                jj = t // 8
                kk = t % 8
                v = sidx_v[jj, 0, pl.ds(kk * 16, 16)]
                rel = v - qbase
                ok = (rel >= 0) & (rel < QROWS)
                sidx_v[jj, 0, pl.ds(kk * 16, 16)] = jnp.where(
                    ok, rel, QROWS)
                return None
            lax.fori_loop(0, 32, remap, None)

            def seg(j, _):
                pltpu.sync_copy(rows_v.at[pl.ds(j * 128, 128)],
                                acc.at[sidx_v.at[j, 0]], add=True)
                return None
            lax.fori_loop(0, 4, seg, None)
            return None
        lax.fori_loop(0, e_per_t // 512, chunk, None)

        plsc.subcore_barrier()
        rpt = QROWS // NS
        def wout(i, _):
            pltpu.sync_copy(acc.at[pl.ds(s * rpt + i * 128, 128)],
                            out.at[pl.ds(qbase + s * rpt + i * 128, 128)])
            return None
        lax.fori_loop(0, rpt // 128, wout, None)
        plsc.subcore_barrier()
        return None

    lax.fori_loop(0, 2, quarter, None)


def _edge_agg(hidden, gidx, sidx3):
    f = pl.kernel(
        _edge_agg_body,
        out_type=jax.ShapeDtypeStruct((NNODES, D), jnp.float32),
        mesh=_MESH,
        compiler_params=pltpu.CompilerParams(use_tc_tiling_on_sc=False),
        scratch_types=[
            pltpu.VMEM((512,), jnp.int32),
            pltpu.VMEM((4, 1, 128), jnp.int32),
            pltpu.VMEM((512, D), jnp.float32),
            pltpu.VMEM((128, D), jnp.float32),
            pltpu.VMEM_SHARED((ACC_ROWS, D), jnp.float32),
            pltpu.SemaphoreType.DMA,
        ],
    )
    return f(hidden, gidx, sidx3)


def _seq_gather_body(hidden, alias_flat, seq, idx_v, rows_v, sem):
    c = lax.axis_index("c")
    s = lax.axis_index("s")
    wid = s * NC + c
    per_w = (NB * LSEQ) // NW

    def chunk(k, _):
        base = wid * per_w + k * 800
        pltpu.sync_copy(alias_flat.at[pl.ds(base, 800)], idx_v)
        pltpu.async_copy(hidden.at[idx_v], rows_v, sem).wait()
        pltpu.sync_copy(rows_v, seq.at[pl.ds(base, 800)])
        return None
    lax.fori_loop(0, per_w // 800, chunk, None)


def _seq_gather(hidden, alias_flat):
    f = pl.kernel(
        _seq_gather_body,
        out_type=jax.ShapeDtypeStruct((NB * LSEQ, D), jnp.float32),
        mesh=_MESH,
        compiler_params=pltpu.CompilerParams(use_tc_tiling_on_sc=False),
        scratch_types=[
            pltpu.VMEM((800,), jnp.int32),
            pltpu.VMEM((800, D), jnp.float32),
            pltpu.SemaphoreType.DMA,
        ],
    )
    return f(hidden, alias_flat)


TCB = 2048
NBLK = NNODES // TCB


def _star0_body(part_ref, cntb_ref, out_ref):
    ssum = part_ref[0] + part_ref[1]
    c = jnp.maximum(cntb_ref[0] + cntb_ref[1], 1.0)
    out_ref[...] = ssum / c[:, None]


def _star0_final(star_part, cnt_b):
    return pl.pallas_call(
        _star0_body,
        out_shape=jax.ShapeDtypeStruct((NB, D), jnp.float32),
    )(star_part, cnt_b)


def _gru_body(hid_ref, ain_ref, aout_ref, cin_ref, cout_ref, batch_ref,
              star_ref, Ain_ref, Aout_ref, Whh_ref, bih_ref, bhh_ref,
              cin_w_ref, cout_w_ref, hout_ref, star_out_ref,
              num_acc, den_acc):
    i = pl.program_id(0)

    @pl.when(i == 0)
    def _zero():
        num_acc[...] = jnp.zeros_like(num_acc)
        den_acc[...] = jnp.zeros_like(den_acc)

    hidden = hid_ref[...]
    cin = cin_ref[0] + cin_ref[1]
    cout = cout_ref[0] + cout_ref[1]
    rin = 1.0 / jnp.maximum(cin, 1.0)
    rout = 1.0 / jnp.maximum(cout, 1.0)
    mi = (cin > 0).astype(jnp.float32)
    mo = (cout > 0).astype(jnp.float32)
    m_in = ain_ref[...] * rin[:, None]
    m_out = aout_ref[...] * rout[:, None]
    gi = (jnp.dot(m_in, Ain_ref[...], preferred_element_type=jnp.float32)
          + jnp.dot(m_out, Aout_ref[...], preferred_element_type=jnp.float32)
          + bih_ref[...][None, :]
          + mi[:, None] * cin_w_ref[...][None, :]
          + mo[:, None] * cout_w_ref[...][None, :])
    gh = (jnp.dot(hidden, Whh_ref[...], preferred_element_type=jnp.float32)
          + bhh_ref[...][None, :])
    i_r, i_i, i_n = gi[:, :D], gi[:, D:2 * D], gi[:, 2 * D:]
    h_r, h_i, h_n = gh[:, :D], gh[:, D:2 * D], gh[:, 2 * D:]
    reset_gate = jax.nn.sigmoid(i_r + h_r)
    input_gate = jax.nn.sigmoid(i_i + h_i)
    new_gate = jnp.tanh(i_n + reset_gate * h_n)
    h1 = (1.0 - input_gate) * hidden + input_gate * new_gate

    bvec = batch_ref[...][:, 0]
    onehot = (bvec[:, None]
              == lax.broadcasted_iota(jnp.int32, (TCB, NB), 1)
              ).astype(jnp.float32)
    star_rep = jnp.dot(onehot, star_ref[...],
                       preferred_element_type=jnp.float32)
    sim = jnp.sum(h1 * star_rep, -1, keepdims=True) * (1.0 / np.sqrt(D))
    alpha = jax.nn.sigmoid(sim)
    h2 = (1.0 - alpha) * h1 + alpha * star_rep
    hout_ref[...] = h2

    s = jnp.sum(h2 * star_rep, -1)
    e = jnp.exp(s)
    dn = (((0,), (0,)), ((), ()))
    den_acc[...] += lax.dot_general(onehot, e[:, None], dn,
                                    preferred_element_type=jnp.float32)
    num_acc[...] += lax.dot_general(onehot, e[:, None] * h2, dn,
                                    preferred_element_type=jnp.float32)

    @pl.when(i == NBLK - 1)
    def _fin():
        star_out_ref[...] = num_acc[...] / jnp.maximum(den_acc[...], 1e-30)


def _gru_step(hidden, agg_in, agg_out, cnt_in2, cnt_out2, batch2, star,
              Ain, Aout, Whh, bih, bhh, cin_w, cout_w):
    blk = lambda i: (i, 0)
    return pl.pallas_call(
        _gru_body,
        grid=(NBLK,),
        in_specs=[
            pl.BlockSpec((TCB, D), blk),
            pl.BlockSpec((TCB, D), blk),
            pl.BlockSpec((TCB, D), blk),
            pl.BlockSpec((2, TCB), lambda i: (0, i)),
            pl.BlockSpec((2, TCB), lambda i: (0, i)),
            pl.BlockSpec((TCB, 1), blk),
            pl.BlockSpec((NB, D), lambda i: (0, 0)),
            pl.BlockSpec((D, 3 * D), lambda i: (0, 0)),
            pl.BlockSpec((D, 3 * D), lambda i: (0, 0)),
            pl.BlockSpec((D, 3 * D), lambda i: (0, 0)),
            pl.BlockSpec((3 * D,), lambda i: (0,)),
            pl.BlockSpec((3 * D,), lambda i: (0,)),
            pl.BlockSpec((3 * D,), lambda i: (0,)),
            pl.BlockSpec((3 * D,), lambda i: (0,)),
        ],
        out_specs=[
            pl.BlockSpec((TCB, D), blk),
            pl.BlockSpec((NB, D), lambda i: (0, 0)),
        ],
        out_shape=[
            jax.ShapeDtypeStruct((NNODES, D), jnp.float32),
            jax.ShapeDtypeStruct((NB, D), jnp.float32),
        ],
        scratch_shapes=[
            pltpu.VMEM((NB, D), jnp.float32),
            pltpu.VMEM((NB, 1), jnp.float32),
        ],
    )(hidden, agg_in, agg_out, cnt_in2, cnt_out2, batch2, star,
      Ain, Aout, Whh, bih, bhh, cin_w, cout_w)


LCH = 8
LBLK = LSEQ // LCH


def _ht_body(seq_ref, isl_ref, pos_ref, ht_ref, ht_acc):
    i = pl.program_id(0)

    @pl.when(i == 0)
    def _zero():
        ht_acc[...] = jnp.zeros_like(ht_acc)

    isl = jnp.maximum(isl_ref[...], 1)
    for j in range(LCH):
        l = i * LCH + j
        shl = seq_ref[:, j, :] + pos_ref[j, :][None, :] * (
            l < isl_ref[...]).astype(jnp.float32)
        ht_acc[...] += shl * (isl - 1 == l).astype(jnp.float32)

    @pl.when(i == LBLK - 1)
    def _fin():
        ht_ref[...] = ht_acc[...]


def _ht_kernel(seq, isl2, pos_emb):
    return pl.pallas_call(
        _ht_body,
        grid=(LBLK,),
        in_specs=[
            pl.BlockSpec((NB, LCH, D), lambda i: (0, i, 0)),
            pl.BlockSpec((NB, 1), lambda i: (0, 0)),
            pl.BlockSpec((LCH, D), lambda i: (i, 0)),
        ],
        out_specs=pl.BlockSpec((NB, D), lambda i: (0, 0)),
        out_shape=jax.ShapeDtypeStruct((NB, D), jnp.float32),
        scratch_shapes=[pltpu.VMEM((NB, D), jnp.float32)],
    )(seq, isl2, pos_emb)


def _readout_body(seq_ref, alias_ref, isl_ref, pos_ref, ht_ref, star_ref,
                  W1_ref, b1_ref, W2_ref, b2_ref, W3_ref, b3_ref, W4_ref,
                  Wta_ref, Wth_ref, bt_ref, out_ref, a_acc):
    i = pl.program_id(0)

    @pl.when(i == 0)
    def _zero():
        a_acc[...] = jnp.zeros_like(a_acc)

    ht = ht_ref[...]
    q1 = jnp.dot(ht, W1_ref[...], preferred_element_type=jnp.float32) \
        + b1_ref[...][None, :]
    q3 = jnp.dot(star_ref[...], W3_ref[...],
                 preferred_element_type=jnp.float32) + b3_ref[...][None, :]
    q13 = q1 + q3
    for j in range(LCH):
        l = i * LCH + j
        shl = seq_ref[:, j, :] + pos_ref[j, :][None, :] * (
            l < isl_ref[...]).astype(jnp.float32)
        q2 = jnp.dot(shl, W2_ref[...], preferred_element_type=jnp.float32) \
            + b2_ref[...][None, :]
        al = jnp.dot(jax.nn.sigmoid(q13 + q2), W4_ref[...],
                     preferred_element_type=jnp.float32)
        msk = (alias_ref[:, j, :] > 0).astype(jnp.float32)
        a_acc[...] += al * shl * msk

    @pl.when(i == LBLK - 1)
    def _fin():
        a = a_acc[...]
        out = (jnp.dot(a, Wta_ref[...], preferred_element_type=jnp.float32)
               + jnp.dot(ht, Wth_ref[...], preferred_element_type=jnp.float32)
               + bt_ref[...][None, :])
        y = out - jnp.mean(out, -1, keepdims=True)
        out_ref[...] = y / jnp.sqrt(jnp.sum(y * y, -1, keepdims=True))


def _readout(seq, alias, isl2, pos_emb, ht, star, p):
    return pl.pallas_call(
        _readout_body,
        grid=(LBLK,),
        in_specs=[
            pl.BlockSpec((NB, LCH, D), lambda i: (0, i, 0)),
            pl.BlockSpec((NB, LCH, 1), lambda i: (0, i, 0)),
            pl.BlockSpec((NB, 1), lambda i: (0, 0)),
            pl.BlockSpec((LCH, D), lambda i: (i, 0)),
            pl.BlockSpec((NB, D), lambda i: (0, 0)),
            pl.BlockSpec((NB, D), lambda i: (0, 0)),
            pl.BlockSpec((D, D), lambda i: (0, 0)),
            pl.BlockSpec((D,), lambda i: (0,)),
            pl.BlockSpec((D, D), lambda i: (0, 0)),
            pl.BlockSpec((D,), lambda i: (0,)),
            pl.BlockSpec((D, D), lambda i: (0, 0)),
            pl.BlockSpec((D,), lambda i: (0,)),
            pl.BlockSpec((D, 1), lambda i: (0, 0)),
            pl.BlockSpec((D, D), lambda i: (0, 0)),
            pl.BlockSpec((D, D), lambda i: (0, 0)),
            pl.BlockSpec((D,), lambda i: (0,)),
        ],
        out_specs=pl.BlockSpec((NB, D), lambda i: (0, 0)),
        out_shape=jax.ShapeDtypeStruct((NB, D), jnp.float32),
        scratch_shapes=[pltpu.VMEM((NB, D), jnp.float32)],
    )(seq, alias, isl2, pos_emb, ht, star,
      p['W1'], p['b1'], p['W2'], p['b2'], p['W3'], p['b3'], p['W4'],
      p['Wt'][:D], p['Wt'][D:], p['bt'])


def kernel(params, x, edge_index, batch, alias_inputs, item_seq_len):
    p = params
    src, dst = edge_index[0], edge_index[1]
    src3 = src.reshape(NE // 128, 1, 128)
    dst3 = dst.reshape(NE // 128, 1, 128)
    batch3 = batch.reshape(NNODES // 128, 1, 128)
    batch2 = batch.reshape(NNODES, 1)

    Ain = p['W_in'] @ p['W_ih'][:D]
    Aout = p['W_out'] @ p['W_ih'][D:]
    cin_w = p['b_in'] @ p['W_ih'][:D]
    cout_w = p['b_out'] @ p['W_ih'][D:]
    bih = p['b_ih']

    hidden, star_part = _emb_gather(p['item_emb'], x, batch3)
    cnt_out2, cnt_in2, cnt_b2 = _hist(src3, dst3, batch3)
    star = _star0_final(star_part, cnt_b2)

    for _ in range(STEP):
        agg_in = _edge_agg(hidden, src, dst3)
        agg_out = _edge_agg(hidden, dst, src3)
        hidden, star = _gru_step(hidden, agg_in, agg_out, cnt_in2, cnt_out2,
                                 batch2, star, Ain, Aout, p['W_hh'], bih,
                                 p['b_hh'], cin_w, cout_w)

    seq = _seq_gather(hidden, alias_inputs.reshape(-1)).reshape(NB, LSEQ, D)
    isl2 = item_seq_len.reshape(NB, 1)
    ht = _ht_kernel(seq, isl2, p['pos_emb'])
    return _readout(seq, alias_inputs.reshape(NB, LSEQ, 1), isl2,
                    p['pos_emb'], ht, star, p)


# speedup vs baseline: 2.3014x; 1.1555x over previous
import functools

import jax
import jax.numpy as jnp
import numpy as np
from jax import lax
from jax.experimental import pallas as pl
from jax.experimental.pallas import tpu as pltpu
from jax.experimental.pallas import tpu_sc as plsc

D = 64
NI = 1000001
LSEQ = 200
NNODES = 65536
NE = 131072
NB = 1024
STEP = 2

NC = 2
NS = 16
NW = NC * NS

_MESH = plsc.VectorSubcoreMesh(core_axis_name="c", subcore_axis_name="s")

QROWS = NNODES // 4
ACC_ROWS = QROWS + 128
ROWS_PER_TILE_Z = ACC_ROWS // NS


def _zero_vmem_rows(buf, nrows):
    zv = jnp.zeros((16,), jnp.float32)

    def body2(i, _):
        r = i // (D // 16)
        k = i % (D // 16)
        buf[r, pl.ds(k * 16, 16)] = zv
        return None

    lax.fori_loop(0, nrows * D // 16, body2, None)


def _emb_gather_body(item_emb, x, batch3, hidden, star_part, idx_v, rows_v,
                     bidx_v, zbuf, star_acc, sem):
    c = lax.axis_index("c")
    s = lax.axis_index("s")
    wid = s * NC + c

    _zero_vmem_rows(zbuf, 128)
    pltpu.sync_copy(zbuf.at[pl.ds(0, 68)], star_acc.at[pl.ds(s * 68, 68)])
    plsc.subcore_barrier()

    rows_per_w = NNODES // NW
    def chunk(k, _):
        base = wid * rows_per_w + k * 1024
        pltpu.sync_copy(x.at[pl.ds(base, 1024)], idx_v)
        pltpu.async_copy(item_emb.at[idx_v], rows_v, sem).wait()
        pltpu.sync_copy(rows_v, hidden.at[pl.ds(base, 1024)])
        pltpu.sync_copy(batch3.at[pl.ds(base // 128, 8)], bidx_v)

        def seg(j, _):
            pltpu.sync_copy(rows_v.at[pl.ds(j * 128, 128)],
                            star_acc.at[bidx_v.at[j, 0]], add=True)
            return None
        lax.fori_loop(0, 8, seg, None)
        return None
    lax.fori_loop(0, 2, chunk, None)

    plsc.subcore_barrier()
    pltpu.sync_copy(star_acc.at[pl.ds(s * 64, 64)],
                    star_part.at[c, pl.ds(s * 64, 64)])


def _emb_gather(item_emb, x, batch3):
    f = pl.kernel(
        _emb_gather_body,
        out_type=[
            jax.ShapeDtypeStruct((NNODES, D), jnp.float32),
            jax.ShapeDtypeStruct((NC, NB, D), jnp.float32),
        ],
        mesh=_MESH,
        compiler_params=pltpu.CompilerParams(use_tc_tiling_on_sc=False),
        scratch_types=[
            pltpu.VMEM((1024,), jnp.int32),
            pltpu.VMEM((1024, D), jnp.float32),
            pltpu.VMEM((8, 1, 128), jnp.int32),
            pltpu.VMEM((128, D), jnp.float32),
            pltpu.VMEM_SHARED((NS * 68, D), jnp.float32),
            pltpu.SemaphoreType.DMA,
        ],
    )
    return f(item_emb, x, batch3)


def _hist_body(src3, dst3, batch3, cnt_out, cnt_in, cnt_b,
               idx_v, ones_v, zflat, hsrc, hdst, hb):
    c = lax.axis_index("c")
    s = lax.axis_index("s")

    def fill16(i, _):
        ones_v[pl.ds(i * 16, 16)] = jnp.ones((16,), jnp.float32)
        return None
    lax.fori_loop(0, 8, fill16, None)
    zv = jnp.zeros((16,), jnp.float32)
    def z16(i, _):
        zflat[pl.ds(i * 16, 16)] = zv
        return None
    lax.fori_loop(0, 4096 // 16, z16, None)
    pltpu.sync_copy(zflat.at[pl.ds(0, 4096)], hsrc.at[pl.ds(s * 4096, 4096)])
    pltpu.sync_copy(zflat.at[pl.ds(0, 4096)], hdst.at[pl.ds(s * 4096, 4096)])
    pltpu.sync_copy(zflat.at[pl.ds(0, 64)], hb.at[pl.ds(s * 64, 64)])
    plsc.subcore_barrier()

    e_per_t = NE // NC // NS
    n_per_t = NNODES // NC // NS

    def do_hist(arr3, acc, per_t, base_rows):
        def chunk(k, _):
            row = base_rows + k
            pltpu.sync_copy(arr3.at[pl.ds(row, 1)], idx_v)
            pltpu.sync_copy(ones_v, acc.at[idx_v.at[0, 0]], add=True)
            return None
        lax.fori_loop(0, per_t // 128, chunk, None)

    base_e = (c * NS + s) * (e_per_t // 128)
    base_n = (c * NS + s) * (n_per_t // 128)
    do_hist(src3, hsrc, e_per_t, base_e)
    do_hist(dst3, hdst, e_per_t, base_e)
    do_hist(batch3, hb, n_per_t, base_n)
    plsc.subcore_barrier()

    per = NNODES // NS
    pltpu.sync_copy(hsrc.at[pl.ds(s * per, per)], cnt_out.at[c, pl.ds(s * per, per)])
    pltpu.sync_copy(hdst.at[pl.ds(s * per, per)], cnt_in.at[c, pl.ds(s * per, per)])
    pltpu.sync_copy(hb.at[pl.ds(s * 64, 64)], cnt_b.at[c, pl.ds(s * 64, 64)])


def _hist(src3, dst3, batch3):
    f = pl.kernel(
        _hist_body,
        out_type=[
            jax.ShapeDtypeStruct((NC, NNODES), jnp.float32),
            jax.ShapeDtypeStruct((NC, NNODES), jnp.float32),
            jax.ShapeDtypeStruct((NC, NB), jnp.float32),
        ],
        mesh=_MESH,
        compiler_params=pltpu.CompilerParams(use_tc_tiling_on_sc=False),
        scratch_types=[
            pltpu.VMEM((1, 1, 128), jnp.int32),
            pltpu.VMEM((128,), jnp.float32),
            pltpu.VMEM((4096,), jnp.float32),
            pltpu.VMEM_SHARED((NNODES,), jnp.float32),
            pltpu.VMEM_SHARED((NNODES,), jnp.float32),
            pltpu.VMEM_SHARED((NS * 64,), jnp.float32),
        ],
    )
    return f(src3, dst3, batch3)


def _edge_agg_body(hidden, gidx, sidx3, out, idx_v, sidx_v, rows_v, zbuf,
                   acc, sem):
    c = lax.axis_index("c")
    s = lax.axis_index("s")

    _zero_vmem_rows(zbuf, 128)

    def quarter(qi, _):
        q = c * 2 + qi
        qbase = q * QROWS

        def zr(i, _):
            pltpu.sync_copy(
                zbuf.at[pl.ds(0, 128)],
                acc.at[pl.ds(s * ROWS_PER_TILE_Z + i * 128, 128)])
            return None
        lax.fori_loop(0, ROWS_PER_TILE_Z // 128, zr, None)
        pltpu.sync_copy(
            zbuf.at[pl.ds(0, ROWS_PER_TILE_Z % 128)],
            acc.at[pl.ds(s * ROWS_PER_TILE_Z
                         + (ROWS_PER_TILE_Z // 128) * 128,
                         ROWS_PER_TILE_Z % 128)])
        plsc.subcore_barrier()

        e_per_t = NE // NS
        def chunk(k, _):
            base = s * e_per_t + k * 512
            pltpu.sync_copy(gidx.at[pl.ds(base, 512)], idx_v)
            pltpu.sync_copy(sidx3.at[pl.ds(base // 128, 4)], sidx_v)
            pltpu.async_copy(hidden.at[idx_v], rows_v, sem).wait()

            def remap(t, _):
                jj = t // 8
                kk = t % 8
                v = sidx_v[jj, 0, pl.ds(kk * 16, 16)]
                rel = v - qbase
                ok = (rel >= 0) & (rel < QROWS)
                sidx_v[jj, 0, pl.ds(kk * 16, 16)] = jnp.where(
                    ok, rel, QROWS)
                return None
            lax.fori_loop(0, 32, remap, None)

            def seg(j, _):
                pltpu.sync_copy(rows_v.at[pl.ds(j * 128, 128)],
                                acc.at[sidx_v.at[j, 0]], add=True)
                return None
            lax.fori_loop(0, 4, seg, None)
            return None
        lax.fori_loop(0, e_per_t // 512, chunk, None)

        plsc.subcore_barrier()
        rpt = QROWS // NS
        def wout(i, _):
            pltpu.sync_copy(acc.at[pl.ds(s * rpt + i * 128, 128)],
                            out.at[pl.ds(qbase + s * rpt + i * 128, 128)])
            return None
        lax.fori_loop(0, rpt // 128, wout, None)
        plsc.subcore_barrier()
        return None

    lax.fori_loop(0, 2, quarter, None)


def _edge_agg(hidden, gidx, sidx3):
    f = pl.kernel(
        _edge_agg_body,
        out_type=jax.ShapeDtypeStruct((NNODES, D), jnp.float32),
        mesh=_MESH,
        compiler_params=pltpu.CompilerParams(use_tc_tiling_on_sc=False),
        scratch_types=[
            pltpu.VMEM((512,), jnp.int32),
            pltpu.VMEM((4, 1, 128), jnp.int32),
            pltpu.VMEM((512, D), jnp.float32),
            pltpu.VMEM((128, D), jnp.float32),
            pltpu.VMEM_SHARED((ACC_ROWS, D), jnp.float32),
            pltpu.SemaphoreType.DMA,
        ],
    )
    return f(hidden, gidx, sidx3)


def _seq_gather_body(hidden, alias_flat, seq, idx_v, rows_v, sem):
    c = lax.axis_index("c")
    s = lax.axis_index("s")
    wid = s * NC + c
    per_w = (NB * LSEQ) // NW

    def chunk(k, _):
        base = wid * per_w + k * 800
        pltpu.sync_copy(alias_flat.at[pl.ds(base, 800)], idx_v)
        pltpu.async_copy(hidden.at[idx_v], rows_v, sem).wait()
        pltpu.sync_copy(rows_v, seq.at[pl.ds(base, 800)])
        return None
    lax.fori_loop(0, per_w // 800, chunk, None)


def _seq_gather(hidden, alias_flat):
    f = pl.kernel(
        _seq_gather_body,
        out_type=jax.ShapeDtypeStruct((NB * LSEQ, D), jnp.float32),
        mesh=_MESH,
        compiler_params=pltpu.CompilerParams(use_tc_tiling_on_sc=False),
        scratch_types=[
            pltpu.VMEM((800,), jnp.int32),
            pltpu.VMEM((800, D), jnp.float32),
            pltpu.SemaphoreType.DMA,
        ],
    )
    return f(hidden, alias_flat)


TCB = 2048
NBLK = NNODES // TCB


def _star0_body(part_ref, cntb_ref, out_ref):
    ssum = part_ref[0] + part_ref[1]
    c = jnp.maximum(cntb_ref[0] + cntb_ref[1], 1.0)
    out_ref[...] = ssum / c[:, None]


def _star0_final(star_part, cnt_b):
    return pl.pallas_call(
        _star0_body,
        out_shape=jax.ShapeDtypeStruct((NB, D), jnp.float32),
    )(star_part, cnt_b)


def _gru_body(hid_ref, ain_ref, aout_ref, cin_ref, cout_ref, batch_ref,
              star_ref, Ain_ref, Aout_ref, Whh_ref, bih_ref, bhh_ref,
              cin_w_ref, cout_w_ref, hout_ref, star_out_ref,
              num_acc, den_acc):
    i = pl.program_id(0)

    @pl.when(i == 0)
    def _zero():
        num_acc[...] = jnp.zeros_like(num_acc)
        den_acc[...] = jnp.zeros_like(den_acc)

    hidden = hid_ref[...]
    cin = cin_ref[0] + cin_ref[1]
    cout = cout_ref[0] + cout_ref[1]
    rin = 1.0 / jnp.maximum(cin, 1.0)
    rout = 1.0 / jnp.maximum(cout, 1.0)
    mi = (cin > 0).astype(jnp.float32)
    mo = (cout > 0).astype(jnp.float32)
    m_in = ain_ref[...] * rin[:, None]
    m_out = aout_ref[...] * rout[:, None]
    gi = (jnp.dot(m_in, Ain_ref[...], preferred_element_type=jnp.float32)
          + jnp.dot(m_out, Aout_ref[...], preferred_element_type=jnp.float32)
          + bih_ref[...][None, :]
          + mi[:, None] * cin_w_ref[...][None, :]
          + mo[:, None] * cout_w_ref[...][None, :])
    gh = (jnp.dot(hidden, Whh_ref[...], preferred_element_type=jnp.float32)
          + bhh_ref[...][None, :])
    i_r, i_i, i_n = gi[:, :D], gi[:, D:2 * D], gi[:, 2 * D:]
    h_r, h_i, h_n = gh[:, :D], gh[:, D:2 * D], gh[:, 2 * D:]
    reset_gate = jax.nn.sigmoid(i_r + h_r)
    input_gate = jax.nn.sigmoid(i_i + h_i)
    new_gate = jnp.tanh(i_n + reset_gate * h_n)
    h1 = (1.0 - input_gate) * hidden + input_gate * new_gate

    bvec = batch_ref[...][:, 0]
    onehot = (bvec[:, None]
              == lax.broadcasted_iota(jnp.int32, (TCB, NB), 1)
              ).astype(jnp.bfloat16)
    star_rep = jnp.dot(onehot, star_ref[...].astype(jnp.bfloat16),
                       preferred_element_type=jnp.float32)
    sim = jnp.sum(h1 * star_rep, -1, keepdims=True) * (1.0 / np.sqrt(D))
    alpha = jax.nn.sigmoid(sim)
    h2 = (1.0 - alpha) * h1 + alpha * star_rep
    hout_ref[...] = h2

    s = jnp.sum(h2 * star_rep, -1)
    e = jnp.exp(s)
    dn = (((0,), (0,)), ((), ()))
    den_acc[...] += lax.dot_general(onehot, e[:, None].astype(jnp.bfloat16),
                                    dn, preferred_element_type=jnp.float32)
    num_acc[...] += lax.dot_general(
        onehot, (e[:, None] * h2).astype(jnp.bfloat16), dn,
        preferred_element_type=jnp.float32)

    @pl.when(i == NBLK - 1)
    def _fin():
        star_out_ref[...] = num_acc[...] / jnp.maximum(den_acc[...], 1e-30)


def _gru_step(hidden, agg_in, agg_out, cnt_in2, cnt_out2, batch2, star,
              Ain, Aout, Whh, bih, bhh, cin_w, cout_w):
    blk = lambda i: (i, 0)
    return pl.pallas_call(
        _gru_body,
        grid=(NBLK,),
        in_specs=[
            pl.BlockSpec((TCB, D), blk),
            pl.BlockSpec((TCB, D), blk),
            pl.BlockSpec((TCB, D), blk),
            pl.BlockSpec((2, TCB), lambda i: (0, i)),
            pl.BlockSpec((2, TCB), lambda i: (0, i)),
            pl.BlockSpec((TCB, 1), blk),
            pl.BlockSpec((NB, D), lambda i: (0, 0)),
            pl.BlockSpec((D, 3 * D), lambda i: (0, 0)),
            pl.BlockSpec((D, 3 * D), lambda i: (0, 0)),
            pl.BlockSpec((D, 3 * D), lambda i: (0, 0)),
            pl.BlockSpec((3 * D,), lambda i: (0,)),
            pl.BlockSpec((3 * D,), lambda i: (0,)),
            pl.BlockSpec((3 * D,), lambda i: (0,)),
            pl.BlockSpec((3 * D,), lambda i: (0,)),
        ],
        out_specs=[
            pl.BlockSpec((TCB, D), blk),
            pl.BlockSpec((NB, D), lambda i: (0, 0)),
        ],
        out_shape=[
            jax.ShapeDtypeStruct((NNODES, D), jnp.float32),
            jax.ShapeDtypeStruct((NB, D), jnp.float32),
        ],
        scratch_shapes=[
            pltpu.VMEM((NB, D), jnp.float32),
            pltpu.VMEM((NB, 1), jnp.float32),
        ],
    )(hidden, agg_in, agg_out, cnt_in2, cnt_out2, batch2, star,
      Ain, Aout, Whh, bih, bhh, cin_w, cout_w)


LCH = 8
LBLK = LSEQ // LCH


def _ht_body(seq_ref, isl_ref, pos_ref, ht_ref, ht_acc):
    i = pl.program_id(0)

    @pl.when(i == 0)
    def _zero():
        ht_acc[...] = jnp.zeros_like(ht_acc)

    isl = jnp.maximum(isl_ref[...], 1)
    for j in range(LCH):
        l = i * LCH + j
        shl = seq_ref[pl.ds(j * NB, NB), :] + pos_ref[j, :][None, :] * (
            l < isl_ref[...]).astype(jnp.float32)
        ht_acc[...] += shl * (isl - 1 == l).astype(jnp.float32)

    @pl.when(i == LBLK - 1)
    def _fin():
        ht_ref[...] = ht_acc[...]


def _ht_kernel(seq, isl2, pos_emb):
    return pl.pallas_call(
        _ht_body,
        grid=(LBLK,),
        in_specs=[
            pl.BlockSpec((LCH * NB, D), lambda i: (i, 0)),
            pl.BlockSpec((NB, 1), lambda i: (0, 0)),
            pl.BlockSpec((LCH, D), lambda i: (i, 0)),
        ],
        out_specs=pl.BlockSpec((NB, D), lambda i: (0, 0)),
        out_shape=jax.ShapeDtypeStruct((NB, D), jnp.float32),
        scratch_shapes=[pltpu.VMEM((NB, D), jnp.float32)],
    )(seq, isl2, pos_emb)


def _readout_body(seq_ref, alias_ref, isl_ref, pos_ref, ht_ref, star_ref,
                  W1_ref, b1_ref, W2_ref, b2_ref, W3_ref, b3_ref, W4_ref,
                  Wta_ref, Wth_ref, bt_ref, out_ref, a_acc):
    i = pl.program_id(0)

    @pl.when(i == 0)
    def _zero():
        a_acc[...] = jnp.zeros_like(a_acc)

    ht = ht_ref[...]
    q1 = jnp.dot(ht, W1_ref[...], preferred_element_type=jnp.float32) \
        + b1_ref[...][None, :]
    q3 = jnp.dot(star_ref[...], W3_ref[...],
                 preferred_element_type=jnp.float32) + b3_ref[...][None, :]
    q13 = q1 + q3
    for j in range(LCH):
        l = i * LCH + j
        shl = seq_ref[pl.ds(j * NB, NB), :] + pos_ref[j, :][None, :] * (
            l < isl_ref[...]).astype(jnp.float32)
        q2 = jnp.dot(shl, W2_ref[...], preferred_element_type=jnp.float32) \
            + b2_ref[...][None, :]
        al = jnp.dot(jax.nn.sigmoid(q13 + q2), W4_ref[...],
                     preferred_element_type=jnp.float32)
        msk = (alias_ref[pl.ds(j * NB, NB), :] > 0).astype(jnp.float32)
        a_acc[...] += al * shl * msk

    @pl.when(i == LBLK - 1)
    def _fin():
        a = a_acc[...]
        out = (jnp.dot(a, Wta_ref[...], preferred_element_type=jnp.float32)
               + jnp.dot(ht, Wth_ref[...], preferred_element_type=jnp.float32)
               + bt_ref[...][None, :])
        y = out - jnp.mean(out, -1, keepdims=True)
        out_ref[...] = y / jnp.sqrt(jnp.sum(y * y, -1, keepdims=True))


def _readout(seq, alias, isl2, pos_emb, ht, star, p):
    return pl.pallas_call(
        _readout_body,
        grid=(LBLK,),
        in_specs=[
            pl.BlockSpec((LCH * NB, D), lambda i: (i, 0)),
            pl.BlockSpec((LCH * NB, 1), lambda i: (i, 0)),
            pl.BlockSpec((NB, 1), lambda i: (0, 0)),
            pl.BlockSpec((LCH, D), lambda i: (i, 0)),
            pl.BlockSpec((NB, D), lambda i: (0, 0)),
            pl.BlockSpec((NB, D), lambda i: (0, 0)),
            pl.BlockSpec((D, D), lambda i: (0, 0)),
            pl.BlockSpec((D,), lambda i: (0,)),
            pl.BlockSpec((D, D), lambda i: (0, 0)),
            pl.BlockSpec((D,), lambda i: (0,)),
            pl.BlockSpec((D, D), lambda i: (0, 0)),
            pl.BlockSpec((D,), lambda i: (0,)),
            pl.BlockSpec((D, 1), lambda i: (0, 0)),
            pl.BlockSpec((D, D), lambda i: (0, 0)),
            pl.BlockSpec((D, D), lambda i: (0, 0)),
            pl.BlockSpec((D,), lambda i: (0,)),
        ],
        out_specs=pl.BlockSpec((NB, D), lambda i: (0, 0)),
        out_shape=jax.ShapeDtypeStruct((NB, D), jnp.float32),
        scratch_shapes=[pltpu.VMEM((NB, D), jnp.float32)],
    )(seq, alias, isl2, pos_emb, ht, star,
      p['W1'], p['b1'], p['W2'], p['b2'], p['W3'], p['b3'], p['W4'],
      p['Wt'][:D], p['Wt'][D:], p['bt'])


def kernel(params, x, edge_index, batch, alias_inputs, item_seq_len):
    p = params
    src, dst = edge_index[0], edge_index[1]
    src3 = src.reshape(NE // 128, 1, 128)
    dst3 = dst.reshape(NE // 128, 1, 128)
    batch3 = batch.reshape(NNODES // 128, 1, 128)
    batch2 = batch.reshape(NNODES, 1)

    Ain = p['W_in'] @ p['W_ih'][:D]
    Aout = p['W_out'] @ p['W_ih'][D:]
    cin_w = p['b_in'] @ p['W_ih'][:D]
    cout_w = p['b_out'] @ p['W_ih'][D:]
    bih = p['b_ih']

    hidden, star_part = _emb_gather(p['item_emb'], x, batch3)
    cnt_out2, cnt_in2, cnt_b2 = _hist(src3, dst3, batch3)
    star = _star0_final(star_part, cnt_b2)

    for _ in range(STEP):
        agg_in = _edge_agg(hidden, src, dst3)
        agg_out = _edge_agg(hidden, dst, src3)
        hidden, star = _gru_step(hidden, agg_in, agg_out, cnt_in2, cnt_out2,
                                 batch2, star, Ain, Aout, p['W_hh'], bih,
                                 p['b_hh'], cin_w, cout_w)

    alias_t = alias_inputs.T.reshape(-1)
    seq = _seq_gather(hidden, alias_t)
    isl2 = item_seq_len.reshape(NB, 1)
    ht = _ht_kernel(seq, isl2, p['pos_emb'])
    return _readout(seq, alias_t.reshape(LSEQ * NB, 1), isl2,
                    p['pos_emb'], ht, star, p)


# speedup vs baseline: 2.9048x; 1.2622x over previous
import functools

import jax
import jax.numpy as jnp
import numpy as np
from jax import lax
from jax.experimental import pallas as pl
from jax.experimental.pallas import tpu as pltpu
from jax.experimental.pallas import tpu_sc as plsc

D = 64
NI = 1000001
LSEQ = 200
NNODES = 65536
NE = 131072
NB = 1024
STEP = 2

NC = 2
NS = 16
NW = NC * NS

_MESH = plsc.VectorSubcoreMesh(core_axis_name="c", subcore_axis_name="s")

QROWS = NNODES // 4
ACC_ROWS = QROWS + 128
ROWS_PER_TILE_Z = ACC_ROWS // NS


def _zero_vmem_rows(buf, nrows):
    zv = jnp.zeros((16,), jnp.float32)

    def body2(i, _):
        r = i // (D // 16)
        k = i % (D // 16)
        buf[r, pl.ds(k * 16, 16)] = zv
        return None

    lax.fori_loop(0, nrows * D // 16, body2, None)


def _emb_gather_body(item_emb, x, batch3, hidden, star_part, idx_v, rows_v,
                     bidx_v, zbuf, star_acc, sem):
    c = lax.axis_index("c")
    s = lax.axis_index("s")
    wid = s * NC + c

    _zero_vmem_rows(zbuf, 128)
    pltpu.sync_copy(zbuf.at[pl.ds(0, 68)], star_acc.at[pl.ds(s * 68, 68)])
    plsc.subcore_barrier()

    rows_per_w = NNODES // NW
    def chunk(k, _):
        base = wid * rows_per_w + k * 1024
        pltpu.sync_copy(x.at[pl.ds(base, 1024)], idx_v)
        pltpu.async_copy(item_emb.at[idx_v], rows_v, sem).wait()
        pltpu.sync_copy(rows_v, hidden.at[pl.ds(base, 1024)])
        pltpu.sync_copy(batch3.at[pl.ds(base // 128, 8)], bidx_v)

        def seg(j, _):
            pltpu.sync_copy(rows_v.at[pl.ds(j * 128, 128)],
                            star_acc.at[bidx_v.at[j, 0]], add=True)
            return None
        lax.fori_loop(0, 8, seg, None)
        return None
    lax.fori_loop(0, 2, chunk, None)

    plsc.subcore_barrier()
    pltpu.sync_copy(star_acc.at[pl.ds(s * 64, 64)],
                    star_part.at[c, pl.ds(s * 64, 64)])


def _emb_gather(item_emb, x, batch3):
    f = pl.kernel(
        _emb_gather_body,
        out_type=[
            jax.ShapeDtypeStruct((NNODES, D), jnp.float32),
            jax.ShapeDtypeStruct((NC, NB, D), jnp.float32),
        ],
        mesh=_MESH,
        compiler_params=pltpu.CompilerParams(use_tc_tiling_on_sc=False),
        scratch_types=[
            pltpu.VMEM((1024,), jnp.int32),
            pltpu.VMEM((1024, D), jnp.float32),
            pltpu.VMEM((8, 1, 128), jnp.int32),
            pltpu.VMEM((128, D), jnp.float32),
            pltpu.VMEM_SHARED((NS * 68, D), jnp.float32),
            pltpu.SemaphoreType.DMA,
        ],
    )
    return f(item_emb, x, batch3)


def _hist_body(src3, dst3, batch3, cnt_out, cnt_in, cnt_b,
               idx_v, ones_v, zflat, hsrc, hdst, hb):
    c = lax.axis_index("c")
    s = lax.axis_index("s")

    def fill16(i, _):
        ones_v[pl.ds(i * 16, 16)] = jnp.ones((16,), jnp.float32)
        return None
    lax.fori_loop(0, 8, fill16, None)
    zv = jnp.zeros((16,), jnp.float32)
    def z16(i, _):
        zflat[pl.ds(i * 16, 16)] = zv
        return None
    lax.fori_loop(0, 4096 // 16, z16, None)
    pltpu.sync_copy(zflat.at[pl.ds(0, 4096)], hsrc.at[pl.ds(s * 4096, 4096)])
    pltpu.sync_copy(zflat.at[pl.ds(0, 4096)], hdst.at[pl.ds(s * 4096, 4096)])
    pltpu.sync_copy(zflat.at[pl.ds(0, 64)], hb.at[pl.ds(s * 64, 64)])
    plsc.subcore_barrier()

    e_per_t = NE // NC // NS
    n_per_t = NNODES // NC // NS

    def do_hist(arr3, acc, per_t, base_rows):
        def chunk(k, _):
            row = base_rows + k
            pltpu.sync_copy(arr3.at[pl.ds(row, 1)], idx_v)
            pltpu.sync_copy(ones_v, acc.at[idx_v.at[0, 0]], add=True)
            return None
        lax.fori_loop(0, per_t // 128, chunk, None)

    base_e = (c * NS + s) * (e_per_t // 128)
    base_n = (c * NS + s) * (n_per_t // 128)
    do_hist(src3, hsrc, e_per_t, base_e)
    do_hist(dst3, hdst, e_per_t, base_e)
    do_hist(batch3, hb, n_per_t, base_n)
    plsc.subcore_barrier()

    per = NNODES // NS
    pltpu.sync_copy(hsrc.at[pl.ds(s * per, per)], cnt_out.at[c, pl.ds(s * per, per)])
    pltpu.sync_copy(hdst.at[pl.ds(s * per, per)], cnt_in.at[c, pl.ds(s * per, per)])
    pltpu.sync_copy(hb.at[pl.ds(s * 64, 64)], cnt_b.at[c, pl.ds(s * 64, 64)])


def _hist(src3, dst3, batch3):
    f = pl.kernel(
        _hist_body,
        out_type=[
            jax.ShapeDtypeStruct((NC, NNODES), jnp.float32),
            jax.ShapeDtypeStruct((NC, NNODES), jnp.float32),
            jax.ShapeDtypeStruct((NC, NB), jnp.float32),
        ],
        mesh=_MESH,
        compiler_params=pltpu.CompilerParams(use_tc_tiling_on_sc=False),
        scratch_types=[
            pltpu.VMEM((1, 1, 128), jnp.int32),
            pltpu.VMEM((128,), jnp.float32),
            pltpu.VMEM((4096,), jnp.float32),
            pltpu.VMEM_SHARED((NNODES,), jnp.float32),
            pltpu.VMEM_SHARED((NNODES,), jnp.float32),
            pltpu.VMEM_SHARED((NS * 64,), jnp.float32),
        ],
    )
    return f(src3, dst3, batch3)


HROWS = NNODES // 2
HACC_ROWS = HROWS + 128
HZ_PER_TILE = HACC_ROWS // NS


def _edge_agg_body(hidden_bf, gidx, sidx3, out, idx_a, idx_b, sidx_a, sidx_b,
                   rows_a, rows_b, zbuf, acc, sem_a, sem_b):
    c = lax.axis_index("c")
    s = lax.axis_index("s")
    hbase = c * HROWS

    zv = jnp.zeros((32,), jnp.bfloat16)
    def zb(i, _):
        r = i // 2
        k = i % 2
        zbuf[r, pl.ds(k * 32, 32)] = zv
        return None
    lax.fori_loop(0, 256, zb, None)

    def zr(i, _):
        pltpu.sync_copy(zbuf.at[pl.ds(0, 128)],
                        acc.at[pl.ds(s * HZ_PER_TILE + i * 128, 128)])
        return None
    lax.fori_loop(0, HZ_PER_TILE // 128, zr, None)
    pltpu.sync_copy(
        zbuf.at[pl.ds(0, HZ_PER_TILE % 128)],
        acc.at[pl.ds(s * HZ_PER_TILE + (HZ_PER_TILE // 128) * 128,
                     HZ_PER_TILE % 128)])
    plsc.subcore_barrier()

    e_per_t = NE // NS
    nch = e_per_t // 512

    def load_idx(ch, idx_v, sidx_v):
        base = s * e_per_t + ch * 512
        pltpu.sync_copy(gidx.at[pl.ds(base, 512)], idx_v)
        pltpu.sync_copy(sidx3.at[pl.ds(base // 128, 4)], sidx_v)

    def consume(sidx_v, rows_v):
        def remap(t, _):
            jj = t // 8
            kk = t % 8
            v = sidx_v[jj, 0, pl.ds(kk * 16, 16)]
            rel = v - hbase
            ok = (rel >= 0) & (rel < HROWS)
            sidx_v[jj, 0, pl.ds(kk * 16, 16)] = jnp.where(ok, rel, HROWS)
            return None
        lax.fori_loop(0, 32, remap, None)

        def seg(j, _):
            pltpu.sync_copy(rows_v.at[pl.ds(j * 128, 128)],
                            acc.at[sidx_v.at[j, 0]], add=True)
            return None
        lax.fori_loop(0, 4, seg, None)

    def pair(k, _):
        load_idx(2 * k, idx_a, sidx_a)
        d_a = pltpu.async_copy(hidden_bf.at[idx_a], rows_a, sem_a)
        load_idx(2 * k + 1, idx_b, sidx_b)
        d_b = pltpu.async_copy(hidden_bf.at[idx_b], rows_b, sem_b)
        d_a.wait()
        consume(sidx_a, rows_a)
        d_b.wait()
        consume(sidx_b, rows_b)
        return None
    lax.fori_loop(0, nch // 2, pair, None)

    plsc.subcore_barrier()
    rpt = HROWS // NS
    def wout(i, _):
        pltpu.sync_copy(acc.at[pl.ds(s * rpt + i * 128, 128)],
                        out.at[pl.ds(hbase + s * rpt + i * 128, 128)])
        return None
    lax.fori_loop(0, rpt // 128, wout, None)


def _edge_agg(hidden_bf, gidx, sidx3):
    f = pl.kernel(
        _edge_agg_body,
        out_type=jax.ShapeDtypeStruct((NNODES, D), jnp.bfloat16),
        mesh=_MESH,
        compiler_params=pltpu.CompilerParams(use_tc_tiling_on_sc=False),
        scratch_types=[
            pltpu.VMEM((512,), jnp.int32),
            pltpu.VMEM((512,), jnp.int32),
            pltpu.VMEM((4, 1, 128), jnp.int32),
            pltpu.VMEM((4, 1, 128), jnp.int32),
            pltpu.VMEM((512, D), jnp.bfloat16),
            pltpu.VMEM((512, D), jnp.bfloat16),
            pltpu.VMEM((128, D), jnp.bfloat16),
            pltpu.VMEM_SHARED((HACC_ROWS, D), jnp.bfloat16),
            pltpu.SemaphoreType.DMA,
            pltpu.SemaphoreType.DMA,
        ],
    )
    return f(hidden_bf, gidx, sidx3)


def _seq_gather_body(hidden, alias_flat, seq, idx_v, rows_v, sem):
    c = lax.axis_index("c")
    s = lax.axis_index("s")
    wid = s * NC + c
    per_w = (NB * LSEQ) // NW

    def chunk(k, _):
        base = wid * per_w + k * 800
        pltpu.sync_copy(alias_flat.at[pl.ds(base, 800)], idx_v)
        pltpu.async_copy(hidden.at[idx_v], rows_v, sem).wait()
        pltpu.sync_copy(rows_v, seq.at[pl.ds(base, 800)])
        return None
    lax.fori_loop(0, per_w // 800, chunk, None)


def _seq_gather(hidden, alias_flat):
    f = pl.kernel(
        _seq_gather_body,
        out_type=jax.ShapeDtypeStruct((NB * LSEQ, D), jnp.float32),
        mesh=_MESH,
        compiler_params=pltpu.CompilerParams(use_tc_tiling_on_sc=False),
        scratch_types=[
            pltpu.VMEM((800,), jnp.int32),
            pltpu.VMEM((800, D), jnp.float32),
            pltpu.SemaphoreType.DMA,
        ],
    )
    return f(hidden, alias_flat)


TCB = 2048
NBLK = NNODES // TCB


def _star0_body(part_ref, cntb_ref, out_ref):
    ssum = part_ref[0] + part_ref[1]
    c = jnp.maximum(cntb_ref[0] + cntb_ref[1], 1.0)
    out_ref[...] = ssum / c[:, None]


def _star0_final(star_part, cnt_b):
    return pl.pallas_call(
        _star0_body,
        out_shape=jax.ShapeDtypeStruct((NB, D), jnp.float32),
    )(star_part, cnt_b)


def _gru_body(hid_ref, ain_ref, aout_ref, cin_ref, cout_ref, batch_ref,
              star_ref, Ain_ref, Aout_ref, Whh_ref, bih_ref, bhh_ref,
              cin_w_ref, cout_w_ref, hout_ref, star_out_ref,
              num_acc, den_acc):
    i = pl.program_id(0)

    @pl.when(i == 0)
    def _zero():
        num_acc[...] = jnp.zeros_like(num_acc)
        den_acc[...] = jnp.zeros_like(den_acc)

    hidden = hid_ref[...]
    cin = cin_ref[0] + cin_ref[1]
    cout = cout_ref[0] + cout_ref[1]
    rin = 1.0 / jnp.maximum(cin, 1.0)
    rout = 1.0 / jnp.maximum(cout, 1.0)
    mi = (cin > 0).astype(jnp.float32)
    mo = (cout > 0).astype(jnp.float32)
    m_in = ain_ref[...].astype(jnp.float32) * rin[:, None]
    m_out = aout_ref[...].astype(jnp.float32) * rout[:, None]
    gi = (jnp.dot(m_in, Ain_ref[...], preferred_element_type=jnp.float32)
          + jnp.dot(m_out, Aout_ref[...], preferred_element_type=jnp.float32)
          + bih_ref[...][None, :]
          + mi[:, None] * cin_w_ref[...][None, :]
          + mo[:, None] * cout_w_ref[...][None, :])
    gh = (jnp.dot(hidden, Whh_ref[...], preferred_element_type=jnp.float32)
          + bhh_ref[...][None, :])
    i_r, i_i, i_n = gi[:, :D], gi[:, D:2 * D], gi[:, 2 * D:]
    h_r, h_i, h_n = gh[:, :D], gh[:, D:2 * D], gh[:, 2 * D:]
    reset_gate = jax.nn.sigmoid(i_r + h_r)
    input_gate = jax.nn.sigmoid(i_i + h_i)
    new_gate = jnp.tanh(i_n + reset_gate * h_n)
    h1 = (1.0 - input_gate) * hidden + input_gate * new_gate

    bvec = batch_ref[...][:, 0]
    onehot = (bvec[:, None]
              == lax.broadcasted_iota(jnp.int32, (TCB, NB), 1)
              ).astype(jnp.bfloat16)
    star_rep = jnp.dot(onehot, star_ref[...].astype(jnp.bfloat16),
                       preferred_element_type=jnp.float32)
    sim = jnp.sum(h1 * star_rep, -1, keepdims=True) * (1.0 / np.sqrt(D))
    alpha = jax.nn.sigmoid(sim)
    h2 = (1.0 - alpha) * h1 + alpha * star_rep
    hout_ref[...] = h2

    s = jnp.sum(h2 * star_rep, -1)
    e = jnp.exp(s)
    dn = (((0,), (0,)), ((), ()))
    den_acc[...] += lax.dot_general(onehot, e[:, None].astype(jnp.bfloat16),
                                    dn, preferred_element_type=jnp.float32)
    num_acc[...] += lax.dot_general(
        onehot, (e[:, None] * h2).astype(jnp.bfloat16), dn,
        preferred_element_type=jnp.float32)

    @pl.when(i == NBLK - 1)
    def _fin():
        star_out_ref[...] = num_acc[...] / jnp.maximum(den_acc[...], 1e-30)


def _gru_step(hidden, agg_in, agg_out, cnt_in2, cnt_out2, batch2, star,
              Ain, Aout, Whh, bih, bhh, cin_w, cout_w):
    blk = lambda i: (i, 0)
    return pl.pallas_call(
        _gru_body,
        grid=(NBLK,),
        in_specs=[
            pl.BlockSpec((TCB, D), blk),
            pl.BlockSpec((TCB, D), blk),
            pl.BlockSpec((TCB, D), blk),
            pl.BlockSpec((2, TCB), lambda i: (0, i)),
            pl.BlockSpec((2, TCB), lambda i: (0, i)),
            pl.BlockSpec((TCB, 1), blk),
            pl.BlockSpec((NB, D), lambda i: (0, 0)),
            pl.BlockSpec((D, 3 * D), lambda i: (0, 0)),
            pl.BlockSpec((D, 3 * D), lambda i: (0, 0)),
            pl.BlockSpec((D, 3 * D), lambda i: (0, 0)),
            pl.BlockSpec((3 * D,), lambda i: (0,)),
            pl.BlockSpec((3 * D,), lambda i: (0,)),
            pl.BlockSpec((3 * D,), lambda i: (0,)),
            pl.BlockSpec((3 * D,), lambda i: (0,)),
        ],
        out_specs=[
            pl.BlockSpec((TCB, D), blk),
            pl.BlockSpec((NB, D), lambda i: (0, 0)),
        ],
        out_shape=[
            jax.ShapeDtypeStruct((NNODES, D), jnp.float32),
            jax.ShapeDtypeStruct((NB, D), jnp.float32),
        ],
        scratch_shapes=[
            pltpu.VMEM((NB, D), jnp.float32),
            pltpu.VMEM((NB, 1), jnp.float32),
        ],
    )(hidden, agg_in, agg_out, cnt_in2, cnt_out2, batch2, star,
      Ain, Aout, Whh, bih, bhh, cin_w, cout_w)


LCH = 8
LBLK = LSEQ // LCH


def _ht_body(seq_ref, isl_ref, pos_ref, ht_ref, ht_acc):
    i = pl.program_id(0)

    @pl.when(i == 0)
    def _zero():
        ht_acc[...] = jnp.zeros_like(ht_acc)

    isl = jnp.maximum(isl_ref[...], 1)
    for j in range(LCH):
        l = i * LCH + j
        shl = seq_ref[pl.ds(j * NB, NB), :] + pos_ref[j, :][None, :] * (
            l < isl_ref[...]).astype(jnp.float32)
        ht_acc[...] += shl * (isl - 1 == l).astype(jnp.float32)

    @pl.when(i == LBLK - 1)
    def _fin():
        ht_ref[...] = ht_acc[...]


def _ht_kernel(seq, isl2, pos_emb):
    return pl.pallas_call(
        _ht_body,
        grid=(LBLK,),
        in_specs=[
            pl.BlockSpec((LCH * NB, D), lambda i: (i, 0)),
            pl.BlockSpec((NB, 1), lambda i: (0, 0)),
            pl.BlockSpec((LCH, D), lambda i: (i, 0)),
        ],
        out_specs=pl.BlockSpec((NB, D), lambda i: (0, 0)),
        out_shape=jax.ShapeDtypeStruct((NB, D), jnp.float32),
        scratch_shapes=[pltpu.VMEM((NB, D), jnp.float32)],
    )(seq, isl2, pos_emb)


def _readout_body(seq_ref, alias_ref, isl_ref, pos_ref, ht_ref, star_ref,
                  W1_ref, b1_ref, W2_ref, b2_ref, W3_ref, b3_ref, W4_ref,
                  Wta_ref, Wth_ref, bt_ref, out_ref, a_acc):
    i = pl.program_id(0)

    @pl.when(i == 0)
    def _zero():
        a_acc[...] = jnp.zeros_like(a_acc)

    ht = ht_ref[...]
    q1 = jnp.dot(ht, W1_ref[...], preferred_element_type=jnp.float32) \
        + b1_ref[...][None, :]
    q3 = jnp.dot(star_ref[...], W3_ref[...],
                 preferred_element_type=jnp.float32) + b3_ref[...][None, :]
    q13 = q1 + q3
    for j in range(LCH):
        l = i * LCH + j
        shl = seq_ref[pl.ds(j * NB, NB), :] + pos_ref[j, :][None, :] * (
            l < isl_ref[...]).astype(jnp.float32)
        q2 = jnp.dot(shl, W2_ref[...], preferred_element_type=jnp.float32) \
            + b2_ref[...][None, :]
        al = jnp.dot(jax.nn.sigmoid(q13 + q2), W4_ref[...],
                     preferred_element_type=jnp.float32)
        msk = (alias_ref[pl.ds(j * NB, NB), :] > 0).astype(jnp.float32)
        a_acc[...] += al * shl * msk

    @pl.when(i == LBLK - 1)
    def _fin():
        a = a_acc[...]
        out = (jnp.dot(a, Wta_ref[...], preferred_element_type=jnp.float32)
               + jnp.dot(ht, Wth_ref[...], preferred_element_type=jnp.float32)
               + bt_ref[...][None, :])
        y = out - jnp.mean(out, -1, keepdims=True)
        out_ref[...] = y / jnp.sqrt(jnp.sum(y * y, -1, keepdims=True))


def _readout(seq, alias, isl2, pos_emb, ht, star, p):
    return pl.pallas_call(
        _readout_body,
        grid=(LBLK,),
        in_specs=[
            pl.BlockSpec((LCH * NB, D), lambda i: (i, 0)),
            pl.BlockSpec((LCH * NB, 1), lambda i: (i, 0)),
            pl.BlockSpec((NB, 1), lambda i: (0, 0)),
            pl.BlockSpec((LCH, D), lambda i: (i, 0)),
            pl.BlockSpec((NB, D), lambda i: (0, 0)),
            pl.BlockSpec((NB, D), lambda i: (0, 0)),
            pl.BlockSpec((D, D), lambda i: (0, 0)),
            pl.BlockSpec((D,), lambda i: (0,)),
            pl.BlockSpec((D, D), lambda i: (0, 0)),
            pl.BlockSpec((D,), lambda i: (0,)),
            pl.BlockSpec((D, D), lambda i: (0, 0)),
            pl.BlockSpec((D,), lambda i: (0,)),
            pl.BlockSpec((D, 1), lambda i: (0, 0)),
            pl.BlockSpec((D, D), lambda i: (0, 0)),
            pl.BlockSpec((D, D), lambda i: (0, 0)),
            pl.BlockSpec((D,), lambda i: (0,)),
        ],
        out_specs=pl.BlockSpec((NB, D), lambda i: (0, 0)),
        out_shape=jax.ShapeDtypeStruct((NB, D), jnp.float32),
        scratch_shapes=[pltpu.VMEM((NB, D), jnp.float32)],
    )(seq, alias, isl2, pos_emb, ht, star,
      p['W1'], p['b1'], p['W2'], p['b2'], p['W3'], p['b3'], p['W4'],
      p['Wt'][:D], p['Wt'][D:], p['bt'])


def kernel(params, x, edge_index, batch, alias_inputs, item_seq_len):
    p = params
    src, dst = edge_index[0], edge_index[1]
    src3 = src.reshape(NE // 128, 1, 128)
    dst3 = dst.reshape(NE // 128, 1, 128)
    batch3 = batch.reshape(NNODES // 128, 1, 128)
    batch2 = batch.reshape(NNODES, 1)

    Ain = p['W_in'] @ p['W_ih'][:D]
    Aout = p['W_out'] @ p['W_ih'][D:]
    cin_w = p['b_in'] @ p['W_ih'][:D]
    cout_w = p['b_out'] @ p['W_ih'][D:]
    bih = p['b_ih']

    hidden, star_part = _emb_gather(p['item_emb'], x, batch3)
    cnt_out2, cnt_in2, cnt_b2 = _hist(src3, dst3, batch3)
    star = _star0_final(star_part, cnt_b2)

    for _ in range(STEP):
        hidden_bf = hidden.astype(jnp.bfloat16)
        agg_in = _edge_agg(hidden_bf, src, dst3)
        agg_out = _edge_agg(hidden_bf, dst, src3)
        hidden, star = _gru_step(hidden, agg_in, agg_out, cnt_in2, cnt_out2,
                                 batch2, star, Ain, Aout, p['W_hh'], bih,
                                 p['b_hh'], cin_w, cout_w)

    alias_t = alias_inputs.T.reshape(-1)
    seq = _seq_gather(hidden, alias_t)
    isl2 = item_seq_len.reshape(NB, 1)
    ht = _ht_kernel(seq, isl2, p['pos_emb'])
    return _readout(seq, alias_t.reshape(LSEQ * NB, 1), isl2,
                    p['pos_emb'], ht, star, p)


# speedup vs baseline: 2.9080x; 1.0011x over previous
import functools

import jax
import jax.numpy as jnp
import numpy as np
from jax import lax
from jax.experimental import pallas as pl
from jax.experimental.pallas import tpu as pltpu
from jax.experimental.pallas import tpu_sc as plsc

D = 64
NI = 1000001
LSEQ = 200
NNODES = 65536
NE = 131072
NB = 1024
STEP = 2

NC = 2
NS = 16
NW = NC * NS

_MESH = plsc.VectorSubcoreMesh(core_axis_name="c", subcore_axis_name="s")

QROWS = NNODES // 4
ACC_ROWS = QROWS + 128
ROWS_PER_TILE_Z = ACC_ROWS // NS


def _zero_vmem_rows(buf, nrows):
    zv = jnp.zeros((16,), jnp.float32)

    def body2(i, _):
        r = i // (D // 16)
        k = i % (D // 16)
        buf[r, pl.ds(k * 16, 16)] = zv
        return None

    lax.fori_loop(0, nrows * D // 16, body2, None)


def _emb_gather_body(item_emb, x, batch3, hidden, star_part, idx_v, rows_v,
                     bidx_v, zbuf, star_acc, sem):
    c = lax.axis_index("c")
    s = lax.axis_index("s")
    wid = s * NC + c

    _zero_vmem_rows(zbuf, 128)
    pltpu.sync_copy(zbuf.at[pl.ds(0, 68)], star_acc.at[pl.ds(s * 68, 68)])
    plsc.subcore_barrier()

    rows_per_w = NNODES // NW
    def chunk(k, _):
        base = wid * rows_per_w + k * 1024
        pltpu.sync_copy(x.at[pl.ds(base, 1024)], idx_v)
        pltpu.async_copy(item_emb.at[idx_v], rows_v, sem).wait()
        pltpu.sync_copy(rows_v, hidden.at[pl.ds(base, 1024)])
        pltpu.sync_copy(batch3.at[pl.ds(base // 128, 8)], bidx_v)

        def seg(j, _):
            pltpu.sync_copy(rows_v.at[pl.ds(j * 128, 128)],
                            star_acc.at[bidx_v.at[j, 0]], add=True)
            return None
        lax.fori_loop(0, 8, seg, None)
        return None
    lax.fori_loop(0, 2, chunk, None)

    plsc.subcore_barrier()
    pltpu.sync_copy(star_acc.at[pl.ds(s * 64, 64)],
                    star_part.at[c, pl.ds(s * 64, 64)])


def _emb_gather(item_emb, x, batch3):
    f = pl.kernel(
        _emb_gather_body,
        out_type=[
            jax.ShapeDtypeStruct((NNODES, D), jnp.float32),
            jax.ShapeDtypeStruct((NC, NB, D), jnp.float32),
        ],
        mesh=_MESH,
        compiler_params=pltpu.CompilerParams(use_tc_tiling_on_sc=False),
        scratch_types=[
            pltpu.VMEM((1024,), jnp.int32),
            pltpu.VMEM((1024, D), jnp.float32),
            pltpu.VMEM((8, 1, 128), jnp.int32),
            pltpu.VMEM((128, D), jnp.float32),
            pltpu.VMEM_SHARED((NS * 68, D), jnp.float32),
            pltpu.SemaphoreType.DMA,
        ],
    )
    return f(item_emb, x, batch3)


def _hist_body(src3, dst3, batch3, cnt_out, cnt_in, cnt_b,
               idx_v, ones_v, zflat, hsrc, hdst, hb):
    c = lax.axis_index("c")
    s = lax.axis_index("s")

    def fill16(i, _):
        ones_v[pl.ds(i * 16, 16)] = jnp.ones((16,), jnp.float32)
        return None
    lax.fori_loop(0, 8, fill16, None)
    zv = jnp.zeros((16,), jnp.float32)
    def z16(i, _):
        zflat[pl.ds(i * 16, 16)] = zv
        return None
    lax.fori_loop(0, 4096 // 16, z16, None)
    pltpu.sync_copy(zflat.at[pl.ds(0, 4096)], hsrc.at[pl.ds(s * 4096, 4096)])
    pltpu.sync_copy(zflat.at[pl.ds(0, 4096)], hdst.at[pl.ds(s * 4096, 4096)])
    pltpu.sync_copy(zflat.at[pl.ds(0, 64)], hb.at[pl.ds(s * 64, 64)])
    plsc.subcore_barrier()

    e_per_t = NE // NC // NS
    n_per_t = NNODES // NC // NS

    def do_hist(arr3, acc, per_t, base_rows):
        def chunk(k, _):
            row = base_rows + k
            pltpu.sync_copy(arr3.at[pl.ds(row, 1)], idx_v)
            pltpu.sync_copy(ones_v, acc.at[idx_v.at[0, 0]], add=True)
            return None
        lax.fori_loop(0, per_t // 128, chunk, None)

    base_e = (c * NS + s) * (e_per_t // 128)
    base_n = (c * NS + s) * (n_per_t // 128)
    do_hist(src3, hsrc, e_per_t, base_e)
    do_hist(dst3, hdst, e_per_t, base_e)
    do_hist(batch3, hb, n_per_t, base_n)
    plsc.subcore_barrier()

    per = NNODES // NS
    pltpu.sync_copy(hsrc.at[pl.ds(s * per, per)], cnt_out.at[c, pl.ds(s * per, per)])
    pltpu.sync_copy(hdst.at[pl.ds(s * per, per)], cnt_in.at[c, pl.ds(s * per, per)])
    pltpu.sync_copy(hb.at[pl.ds(s * 64, 64)], cnt_b.at[c, pl.ds(s * 64, 64)])


def _hist(src3, dst3, batch3):
    f = pl.kernel(
        _hist_body,
        out_type=[
            jax.ShapeDtypeStruct((NC, NNODES), jnp.float32),
            jax.ShapeDtypeStruct((NC, NNODES), jnp.float32),
            jax.ShapeDtypeStruct((NC, NB), jnp.float32),
        ],
        mesh=_MESH,
        compiler_params=pltpu.CompilerParams(use_tc_tiling_on_sc=False),
        scratch_types=[
            pltpu.VMEM((1, 1, 128), jnp.int32),
            pltpu.VMEM((128,), jnp.float32),
            pltpu.VMEM((4096,), jnp.float32),
            pltpu.VMEM_SHARED((NNODES,), jnp.float32),
            pltpu.VMEM_SHARED((NNODES,), jnp.float32),
            pltpu.VMEM_SHARED((NS * 64,), jnp.float32),
        ],
    )
    return f(src3, dst3, batch3)


HROWS = NNODES // 2
HACC_ROWS = HROWS + 128
HZ_PER_TILE = HACC_ROWS // NS


def _edge_agg_body(hidden_bf, gidx, sidx3, out, idx_a, idx_b, sidx_a, sidx_b,
                   rows_a, rows_b, zbuf, acc, sem_a, sem_b):
    c = lax.axis_index("c")
    s = lax.axis_index("s")
    hbase = c * HROWS

    zv = jnp.zeros((32,), jnp.bfloat16)
    def zb(i, _):
        r = i // 2
        k = i % 2
        zbuf[r, pl.ds(k * 32, 32)] = zv
        return None
    lax.fori_loop(0, 256, zb, None)

    def zr(i, _):
        pltpu.sync_copy(zbuf.at[pl.ds(0, 128)],
                        acc.at[pl.ds(s * HZ_PER_TILE + i * 128, 128)])
        return None
    lax.fori_loop(0, HZ_PER_TILE // 128, zr, None)
    pltpu.sync_copy(
        zbuf.at[pl.ds(0, HZ_PER_TILE % 128)],
        acc.at[pl.ds(s * HZ_PER_TILE + (HZ_PER_TILE // 128) * 128,
                     HZ_PER_TILE % 128)])
    plsc.subcore_barrier()

    e_per_t = NE // NS
    nch = e_per_t // 512

    def load_idx(ch, idx_v, sidx_v):
        base = s * e_per_t + ch * 512
        pltpu.sync_copy(gidx.at[pl.ds(base, 512)], idx_v)
        pltpu.sync_copy(sidx3.at[pl.ds(base // 128, 4)], sidx_v)

    def consume(sidx_v, rows_v):
        def remap(t, _):
            jj = t // 8
            kk = t % 8
            v = sidx_v[jj, 0, pl.ds(kk * 16, 16)]
            rel = v - hbase
            ok = (rel >= 0) & (rel < HROWS)
            sidx_v[jj, 0, pl.ds(kk * 16, 16)] = jnp.where(ok, rel, HROWS)
            return None
        lax.fori_loop(0, 32, remap, None)

        def seg(j, _):
            pltpu.sync_copy(rows_v.at[pl.ds(j * 128, 128)],
                            acc.at[sidx_v.at[j, 0]], add=True)
            return None
        lax.fori_loop(0, 4, seg, None)

    def pair(k, _):
        load_idx(2 * k, idx_a, sidx_a)
        d_a = pltpu.async_copy(hidden_bf.at[idx_a], rows_a, sem_a)
        load_idx(2 * k + 1, idx_b, sidx_b)
        d_b = pltpu.async_copy(hidden_bf.at[idx_b], rows_b, sem_b)
        d_a.wait()
        consume(sidx_a, rows_a)
        d_b.wait()
        consume(sidx_b, rows_b)
        return None
    lax.fori_loop(0, nch // 2, pair, None)

    plsc.subcore_barrier()
    rpt = HROWS // NS
    def wout(i, _):
        pltpu.sync_copy(acc.at[pl.ds(s * rpt + i * 128, 128)],
                        out.at[pl.ds(hbase + s * rpt + i * 128, 128)])
        return None
    lax.fori_loop(0, rpt // 128, wout, None)


def _edge_agg(hidden_bf, gidx, sidx3):
    f = pl.kernel(
        _edge_agg_body,
        out_type=jax.ShapeDtypeStruct((NNODES, D), jnp.bfloat16),
        mesh=_MESH,
        compiler_params=pltpu.CompilerParams(use_tc_tiling_on_sc=False),
        scratch_types=[
            pltpu.VMEM((512,), jnp.int32),
            pltpu.VMEM((512,), jnp.int32),
            pltpu.VMEM((4, 1, 128), jnp.int32),
            pltpu.VMEM((4, 1, 128), jnp.int32),
            pltpu.VMEM((512, D), jnp.bfloat16),
            pltpu.VMEM((512, D), jnp.bfloat16),
            pltpu.VMEM((128, D), jnp.bfloat16),
            pltpu.VMEM_SHARED((HACC_ROWS, D), jnp.bfloat16),
            pltpu.SemaphoreType.DMA,
            pltpu.SemaphoreType.DMA,
        ],
    )
    return f(hidden_bf, gidx, sidx3)


def _seq_gather_body(hidden, alias_flat, seq, idx_v, rows_v, sem):
    c = lax.axis_index("c")
    s = lax.axis_index("s")
    wid = s * NC + c
    per_w = (NB * LSEQ) // NW

    def chunk(k, _):
        base = wid * per_w + k * 800
        pltpu.sync_copy(alias_flat.at[pl.ds(base, 800)], idx_v)
        pltpu.async_copy(hidden.at[idx_v], rows_v, sem).wait()
        pltpu.sync_copy(rows_v, seq.at[pl.ds(base, 800)])
        return None
    lax.fori_loop(0, per_w // 800, chunk, None)


def _seq_gather(hidden, alias_flat):
    f = pl.kernel(
        _seq_gather_body,
        out_type=jax.ShapeDtypeStruct((NB * LSEQ, D), jnp.float32),
        mesh=_MESH,
        compiler_params=pltpu.CompilerParams(use_tc_tiling_on_sc=False),
        scratch_types=[
            pltpu.VMEM((800,), jnp.int32),
            pltpu.VMEM((800, D), jnp.float32),
            pltpu.SemaphoreType.DMA,
        ],
    )
    return f(hidden, alias_flat)


TCB = 2048
NBLK = NNODES // TCB


def _star0_body(part_ref, cntb_ref, out_ref):
    ssum = part_ref[0] + part_ref[1]
    c = jnp.maximum(cntb_ref[0] + cntb_ref[1], 1.0)
    out_ref[...] = ssum / c[:, None]


def _star0_final(star_part, cnt_b):
    return pl.pallas_call(
        _star0_body,
        out_shape=jax.ShapeDtypeStruct((NB, D), jnp.float32),
    )(star_part, cnt_b)


def _gru_body(hid_ref, ain_ref, aout_ref, cin_ref, cout_ref, batch_ref,
              star_ref, Ain_ref, Aout_ref, Whh_ref, bih_ref, bhh_ref,
              cin_w_ref, cout_w_ref, hout_ref, star_out_ref,
              num_acc, den_acc):
    i = pl.program_id(0)

    @pl.when(i == 0)
    def _zero():
        num_acc[...] = jnp.zeros_like(num_acc)
        den_acc[...] = jnp.zeros_like(den_acc)

    hidden = hid_ref[...]
    cin = cin_ref[0] + cin_ref[1]
    cout = cout_ref[0] + cout_ref[1]
    rin = 1.0 / jnp.maximum(cin, 1.0)
    rout = 1.0 / jnp.maximum(cout, 1.0)
    mi = (cin > 0).astype(jnp.float32)
    mo = (cout > 0).astype(jnp.float32)
    m_in = ain_ref[...].astype(jnp.float32) * rin[:, None]
    m_out = aout_ref[...].astype(jnp.float32) * rout[:, None]
    gi = (jnp.dot(m_in, Ain_ref[...], preferred_element_type=jnp.float32)
          + jnp.dot(m_out, Aout_ref[...], preferred_element_type=jnp.float32)
          + bih_ref[...][None, :]
          + mi[:, None] * cin_w_ref[...][None, :]
          + mo[:, None] * cout_w_ref[...][None, :])
    gh = (jnp.dot(hidden, Whh_ref[...], preferred_element_type=jnp.float32)
          + bhh_ref[...][None, :])
    i_r, i_i, i_n = gi[:, :D], gi[:, D:2 * D], gi[:, 2 * D:]
    h_r, h_i, h_n = gh[:, :D], gh[:, D:2 * D], gh[:, 2 * D:]
    reset_gate = jax.nn.sigmoid(i_r + h_r)
    input_gate = jax.nn.sigmoid(i_i + h_i)
    new_gate = jnp.tanh(i_n + reset_gate * h_n)
    h1 = (1.0 - input_gate) * hidden + input_gate * new_gate

    bvec = batch_ref[...][:, 0]
    onehot = (bvec[:, None]
              == lax.broadcasted_iota(jnp.int32, (TCB, NB), 1)
              ).astype(jnp.bfloat16)
    star_rep = jnp.dot(onehot, star_ref[...].astype(jnp.bfloat16),
                       preferred_element_type=jnp.float32)
    sim = jnp.sum(h1 * star_rep, -1, keepdims=True) * (1.0 / np.sqrt(D))
    alpha = jax.nn.sigmoid(sim)
    h2 = (1.0 - alpha) * h1 + alpha * star_rep
    hout_ref[...] = h2

    s = jnp.sum(h2 * star_rep, -1)
    e = jnp.exp(s)
    dn = (((0,), (0,)), ((), ()))
    den_acc[...] += lax.dot_general(onehot, e[:, None].astype(jnp.bfloat16),
                                    dn, preferred_element_type=jnp.float32)
    num_acc[...] += lax.dot_general(
        onehot, (e[:, None] * h2).astype(jnp.bfloat16), dn,
        preferred_element_type=jnp.float32)

    @pl.when(i == NBLK - 1)
    def _fin():
        star_out_ref[...] = num_acc[...] / jnp.maximum(den_acc[...], 1e-30)


def _gru_step(hidden, agg_in, agg_out, cnt_in2, cnt_out2, batch2, star,
              Ain, Aout, Whh, bih, bhh, cin_w, cout_w):
    blk = lambda i: (i, 0)
    return pl.pallas_call(
        _gru_body,
        grid=(NBLK,),
        in_specs=[
            pl.BlockSpec((TCB, D), blk),
            pl.BlockSpec((TCB, D), blk),
            pl.BlockSpec((TCB, D), blk),
            pl.BlockSpec((2, TCB), lambda i: (0, i)),
            pl.BlockSpec((2, TCB), lambda i: (0, i)),
            pl.BlockSpec((TCB, 1), blk),
            pl.BlockSpec((NB, D), lambda i: (0, 0)),
            pl.BlockSpec((D, 3 * D), lambda i: (0, 0)),
            pl.BlockSpec((D, 3 * D), lambda i: (0, 0)),
            pl.BlockSpec((D, 3 * D), lambda i: (0, 0)),
            pl.BlockSpec((3 * D,), lambda i: (0,)),
            pl.BlockSpec((3 * D,), lambda i: (0,)),
            pl.BlockSpec((3 * D,), lambda i: (0,)),
            pl.BlockSpec((3 * D,), lambda i: (0,)),
        ],
        out_specs=[
            pl.BlockSpec((TCB, D), blk),
            pl.BlockSpec((NB, D), lambda i: (0, 0)),
        ],
        out_shape=[
            jax.ShapeDtypeStruct((NNODES, D), jnp.float32),
            jax.ShapeDtypeStruct((NB, D), jnp.float32),
        ],
        scratch_shapes=[
            pltpu.VMEM((NB, D), jnp.float32),
            pltpu.VMEM((NB, 1), jnp.float32),
        ],
    )(hidden, agg_in, agg_out, cnt_in2, cnt_out2, batch2, star,
      Ain, Aout, Whh, bih, bhh, cin_w, cout_w)


def _transpose_body(a_ref, out_ref):
    out_ref[...] = jnp.transpose(
        a_ref[...].astype(jnp.float32)).astype(jnp.int32)


def _alias_transpose(alias):
    return pl.pallas_call(
        _transpose_body,
        out_shape=jax.ShapeDtypeStruct((LSEQ, NB), jnp.int32),
    )(alias)


LCH = 8
LBLK = LSEQ // LCH


def _ht_body(seq_ref, isl_ref, pos_ref, ht_ref, ht_acc):
    i = pl.program_id(0)

    @pl.when(i == 0)
    def _zero():
        ht_acc[...] = jnp.zeros_like(ht_acc)

    isl = jnp.maximum(isl_ref[...], 1)
    for j in range(LCH):
        l = i * LCH + j
        shl = seq_ref[pl.ds(j * NB, NB), :] + pos_ref[j, :][None, :] * (
            l < isl_ref[...]).astype(jnp.float32)
        ht_acc[...] += shl * (isl - 1 == l).astype(jnp.float32)

    @pl.when(i == LBLK - 1)
    def _fin():
        ht_ref[...] = ht_acc[...]


def _ht_kernel(seq, isl2, pos_emb):
    return pl.pallas_call(
        _ht_body,
        grid=(LBLK,),
        in_specs=[
            pl.BlockSpec((LCH * NB, D), lambda i: (i, 0)),
            pl.BlockSpec((NB, 1), lambda i: (0, 0)),
            pl.BlockSpec((LCH, D), lambda i: (i, 0)),
        ],
        out_specs=pl.BlockSpec((NB, D), lambda i: (0, 0)),
        out_shape=jax.ShapeDtypeStruct((NB, D), jnp.float32),
        scratch_shapes=[pltpu.VMEM((NB, D), jnp.float32)],
    )(seq, isl2, pos_emb)


def _readout_body(seq_ref, alias_ref, isl_ref, pos_ref, ht_ref, star_ref,
                  W1_ref, b1_ref, W2_ref, b2_ref, W3_ref, b3_ref, W4_ref,
                  Wta_ref, Wth_ref, bt_ref, out_ref, a_acc):
    i = pl.program_id(0)

    @pl.when(i == 0)
    def _zero():
        a_acc[...] = jnp.zeros_like(a_acc)

    ht = ht_ref[...]
    q1 = jnp.dot(ht, W1_ref[...], preferred_element_type=jnp.float32) \
        + b1_ref[...][None, :]
    q3 = jnp.dot(star_ref[...], W3_ref[...],
                 preferred_element_type=jnp.float32) + b3_ref[...][None, :]
    q13 = q1 + q3
    for j in range(LCH):
        l = i * LCH + j
        shl = seq_ref[pl.ds(j * NB, NB), :] + pos_ref[j, :][None, :] * (
            l < isl_ref[...]).astype(jnp.float32)
        q2 = jnp.dot(shl, W2_ref[...], preferred_element_type=jnp.float32) \
            + b2_ref[...][None, :]
        al = jnp.dot(jax.nn.sigmoid(q13 + q2), W4_ref[...],
                     preferred_element_type=jnp.float32)
        msk = (alias_ref[pl.ds(j * NB, NB), :] > 0).astype(jnp.float32)
        a_acc[...] += al * shl * msk

    @pl.when(i == LBLK - 1)
    def _fin():
        a = a_acc[...]
        out = (jnp.dot(a, Wta_ref[...], preferred_element_type=jnp.float32)
               + jnp.dot(ht, Wth_ref[...], preferred_element_type=jnp.float32)
               + bt_ref[...][None, :])
        y = out - jnp.mean(out, -1, keepdims=True)
        out_ref[...] = y / jnp.sqrt(jnp.sum(y * y, -1, keepdims=True))


def _readout(seq, alias, isl2, pos_emb, ht, star, p):
    return pl.pallas_call(
        _readout_body,
        grid=(LBLK,),
        in_specs=[
            pl.BlockSpec((LCH * NB, D), lambda i: (i, 0)),
            pl.BlockSpec((LCH * NB, 1), lambda i: (i, 0)),
            pl.BlockSpec((NB, 1), lambda i: (0, 0)),
            pl.BlockSpec((LCH, D), lambda i: (i, 0)),
            pl.BlockSpec((NB, D), lambda i: (0, 0)),
            pl.BlockSpec((NB, D), lambda i: (0, 0)),
            pl.BlockSpec((D, D), lambda i: (0, 0)),
            pl.BlockSpec((D,), lambda i: (0,)),
            pl.BlockSpec((D, D), lambda i: (0, 0)),
            pl.BlockSpec((D,), lambda i: (0,)),
            pl.BlockSpec((D, D), lambda i: (0, 0)),
            pl.BlockSpec((D,), lambda i: (0,)),
            pl.BlockSpec((D, 1), lambda i: (0, 0)),
            pl.BlockSpec((D, D), lambda i: (0, 0)),
            pl.BlockSpec((D, D), lambda i: (0, 0)),
            pl.BlockSpec((D,), lambda i: (0,)),
        ],
        out_specs=pl.BlockSpec((NB, D), lambda i: (0, 0)),
        out_shape=jax.ShapeDtypeStruct((NB, D), jnp.float32),
        scratch_shapes=[pltpu.VMEM((NB, D), jnp.float32)],
    )(seq, alias, isl2, pos_emb, ht, star,
      p['W1'], p['b1'], p['W2'], p['b2'], p['W3'], p['b3'], p['W4'],
      p['Wt'][:D], p['Wt'][D:], p['bt'])


def kernel(params, x, edge_index, batch, alias_inputs, item_seq_len):
    p = params
    src, dst = edge_index[0], edge_index[1]
    src3 = src.reshape(NE // 128, 1, 128)
    dst3 = dst.reshape(NE // 128, 1, 128)
    batch3 = batch.reshape(NNODES // 128, 1, 128)
    batch2 = batch.reshape(NNODES, 1)

    Ain = p['W_in'] @ p['W_ih'][:D]
    Aout = p['W_out'] @ p['W_ih'][D:]
    cin_w = p['b_in'] @ p['W_ih'][:D]
    cout_w = p['b_out'] @ p['W_ih'][D:]
    bih = p['b_ih']

    hidden, star_part = _emb_gather(p['item_emb'], x, batch3)
    cnt_out2, cnt_in2, cnt_b2 = _hist(src3, dst3, batch3)
    star = _star0_final(star_part, cnt_b2)

    for _ in range(STEP):
        hidden_bf = hidden.astype(jnp.bfloat16)
        agg_in = _edge_agg(hidden_bf, src, dst3)
        agg_out = _edge_agg(hidden_bf, dst, src3)
        hidden, star = _gru_step(hidden, agg_in, agg_out, cnt_in2, cnt_out2,
                                 batch2, star, Ain, Aout, p['W_hh'], bih,
                                 p['b_hh'], cin_w, cout_w)

    alias_t = _alias_transpose(alias_inputs).reshape(-1)
    seq = _seq_gather(hidden, alias_t)
    isl2 = item_seq_len.reshape(NB, 1)
    ht = _ht_kernel(seq, isl2, p['pos_emb'])
    return _readout(seq, alias_t.reshape(LSEQ * NB, 1), isl2,
                    p['pos_emb'], ht, star, p)
